# Initial kernel scaffold; baseline (speedup 1.0000x reference)
import jax
import jax.numpy as jnp
from jax.experimental import pallas as pl


def kernel(x, fine_edges, coarse_edges, params):
    raise NotImplementedError("write your pallas kernel here")


# speedup vs baseline: 1.8207x; 1.8207x over previous
import dataclasses
import functools

import jax
import jax.numpy as jnp
from jax import lax
from jax.experimental import pallas as pl
from jax.experimental.pallas import tpu as pltpu
from jax.experimental.pallas import tpu_sc as plsc

N = 10000
C = 128
H = 128
FH = 6
LN_EPS = 1e-5

BLK = 2000
NPAD = 10112
SW = 128
K = 128
NCORES = 2
NSUB = 16
LANE = 16
NF = H // LANE
SPLIT = N // 2
TPC = 5120
RPT = TPC // NSUB


def _ln(t, g, b):
    mu = jnp.mean(t, axis=-1, keepdims=True)
    var = jnp.mean((t - mu) ** 2, axis=-1, keepdims=True)
    return (t - mu) * lax.rsqrt(var + LN_EPS) * g + b


def _rows(i):
    return (i, 0)


def _const(i):
    return (0, 0)


def _mm(a, b):
    return jnp.dot(a, b, preferred_element_type=jnp.float32)


def _enc_body(x_ref, w1_ref, b1_ref, g_ref, bb_ref, w2_ref, c2_ref, o_ref):
    t = _mm(x_ref[:], w1_ref[:]) + b1_ref[:]
    y = jnp.maximum(_ln(t, g_ref[:], bb_ref[:]), 0.0)
    o_ref[:] = _mm(y, w2_ref[:]) + c2_ref[:]


def _tc_enc(x, w1, b1, g, bb, w2, c2):
    return pl.pallas_call(
        _enc_body,
        grid=(N // BLK,),
        in_specs=[
            pl.BlockSpec((BLK, C), _rows),
            pl.BlockSpec((C, H), _const),
            pl.BlockSpec((1, H), _const),
            pl.BlockSpec((1, H), _const),
            pl.BlockSpec((1, H), _const),
            pl.BlockSpec((H, H), _const),
            pl.BlockSpec((1, H), _const),
        ],
        out_specs=pl.BlockSpec((BLK, H), _rows),
        out_shape=jax.ShapeDtypeStruct((N, H), jnp.float32),
    )(x, w1, b1, g, bb, w2, c2)


def _pre_body(h_ref, wa_ref, wb_ref, b1_ref, g_ref, bb_ref,
              a_ref, b_ref, s_ref):
    h = h_ref[:]
    A = _mm(h, wa_ref[:]) + b1_ref[:]
    B = _mm(h, wb_ref[:])
    a_ref[:] = A
    b_ref[:] = B
    s_ref[:] = jnp.maximum(_ln(A + B, g_ref[:], bb_ref[:]), 0.0)


def _tc_pre(h, wa, wb, b1, g, bb):
    return pl.pallas_call(
        _pre_body,
        grid=(N // BLK,),
        in_specs=[
            pl.BlockSpec((BLK, H), _rows),
            pl.BlockSpec((H, H), _const),
            pl.BlockSpec((H, H), _const),
            pl.BlockSpec((1, H), _const),
            pl.BlockSpec((1, H), _const),
            pl.BlockSpec((1, H), _const),
        ],
        out_specs=[
            pl.BlockSpec((BLK, H), _rows),
            pl.BlockSpec((BLK, H), _rows),
            pl.BlockSpec((BLK, H), _rows),
        ],
        out_shape=[
            jax.ShapeDtypeStruct((N, H), jnp.float32),
            jax.ShapeDtypeStruct((N, H), jnp.float32),
            jax.ShapeDtypeStruct((N, H), jnp.float32),
        ],
    )(h, wa, wb, b1, g, bb)


def _post_body(s_ref, c_ref, ss_ref, h_ref, w2_ref,
               b2_ref, ua_ref, ub_ref, c1_ref, gu_ref, bu_ref, u2_ref,
               c2_ref, o_ref):
    S = s_ref[:] + ss_ref[:]
    cnt = c_ref[:] + 1.0
    aggr = _mm(S, w2_ref[:]) + cnt * b2_ref[:]
    t = _mm(h_ref[:], ua_ref[:]) + _mm(aggr, ub_ref[:]) + c1_ref[:]
    y = jnp.maximum(_ln(t, gu_ref[:], bu_ref[:]), 0.0)
    o_ref[:] = _mm(y, u2_ref[:]) + c2_ref[:]


def _tc_post(s, c, ss, h, w2, b2, ua, ub, c1, gu, bu, u2, c2):
    return pl.pallas_call(
        _post_body,
        grid=(N // BLK,),
        in_specs=[
            pl.BlockSpec((BLK, H), _rows),
            pl.BlockSpec((BLK, 1), _rows),
            pl.BlockSpec((BLK, H), _rows),
            pl.BlockSpec((BLK, H), _rows),
            pl.BlockSpec((H, H), _const),
            pl.BlockSpec((1, H), _const),
            pl.BlockSpec((H, H), _const),
            pl.BlockSpec((H, H), _const),
            pl.BlockSpec((1, H), _const),
            pl.BlockSpec((1, H), _const),
            pl.BlockSpec((1, H), _const),
            pl.BlockSpec((H, H), _const),
            pl.BlockSpec((1, H), _const),
        ],
        out_specs=pl.BlockSpec((BLK, H), _rows),
        out_shape=jax.ShapeDtypeStruct((N, H), jnp.float32),
    )(s, c, ss, h, w2, b2, ua, ub, c1, gu, bu, u2, c2)


def _dec_body(hf_ref, hc_ref, wma_ref, wmb_ref, bm_ref,
              wd1_ref, bd1_ref, wd2_ref, bd2_ref, o_ref):
    comb = _mm(hf_ref[:], wma_ref[:]) + _mm(hc_ref[:], wmb_ref[:]) + bm_ref[:]
    d = jnp.maximum(_mm(comb, wd1_ref[:]) + bd1_ref[:], 0.0)
    o_ref[:] = _mm(d, wd2_ref[:]) + bd2_ref[:]


def _tc_dec(hf, hc, wma, wmb, bm, wd1, bd1, wd2, bd2):
    OD = FH * C
    return pl.pallas_call(
        _dec_body,
        grid=(N // BLK,),
        in_specs=[
            pl.BlockSpec((BLK, H), _rows),
            pl.BlockSpec((BLK, H), _rows),
            pl.BlockSpec((H, H), _const),
            pl.BlockSpec((H, H), _const),
            pl.BlockSpec((1, H), _const),
            pl.BlockSpec((H, H), _const),
            pl.BlockSpec((1, H), _const),
            pl.BlockSpec((H, OD), _const),
            pl.BlockSpec((1, OD), _const),
        ],
        out_specs=pl.BlockSpec((BLK, OD), _rows),
        out_shape=jax.ShapeDtypeStruct((N, OD), jnp.float32),
    )(hf, hc, wma, wmb, bm, wd1, bd1, wd2, bd2)


def _sc_cp():
    cp = pltpu.CompilerParams()
    if "needs_layout_passes" in pltpu.CompilerParams.__dataclass_fields__:
        cp = dataclasses.replace(cp, needs_layout_passes=False)
    return cp


def _make_sc_edge(cpw):
    mesh = plsc.VectorSubcoreMesh(core_axis_name="c", subcore_axis_name="s")

    @functools.partial(
        pl.kernel,
        out_type=jax.ShapeDtypeStruct((NCORES, TPC, SW), jnp.float32),
        mesh=mesh,
        compiler_params=_sc_cp(),
        scratch_types=[
            pltpu.VMEM_SHARED((TPC, SW), jnp.float32),
            pltpu.VMEM((2, K, H), jnp.float32),
            pltpu.VMEM((2, K, H), jnp.float32),
            pltpu.VMEM((2, K), jnp.int32),
            pltpu.VMEM((2, K), jnp.int32),
            pltpu.VMEM((2, K), jnp.int32),
            pltpu.VMEM((2, H), jnp.float32),
            pltpu.SemaphoreType.DMA,
            pltpu.SemaphoreType.DMA,
            pltpu.SemaphoreType.DMA,
        ],
    )
    def kern(ap, bp, dsth, srch, zz, gb, out, s_sh, bufa, bufb,
             dsti, srci, dstl, gbv, sga, sgb, sm):
        cid = lax.axis_index("c")
        sid = lax.axis_index("s")
        base = cid * SPLIT

        pltpu.async_copy(gb, gbv, sm).wait()
        pltpu.async_copy(dsth.at[cid, sid, 0], dsti.at[0], sm).wait()
        pltpu.async_copy(srch.at[cid, sid, 0], srci.at[0], sm).wait()
        pltpu.async_copy(dsth.at[cid, sid, 1], dsti.at[1], sm)
        pltpu.async_copy(srch.at[cid, sid, 1], srci.at[1], sm)
        pltpu.async_copy(zz.at[pl.ds(sid * RPT, RPT)],
                         s_sh.at[pl.ds(sid * RPT, RPT)], sm).wait()
        plsc.subcore_barrier()

        gvec = [gbv[0, pl.ds(f * LANE, LANE)] for f in range(NF)]
        bvec = [gbv[1, pl.ds(f * LANE, LANE)] for f in range(NF)]
        magic = jnp.full((LANE,), 0x5F3759DF, jnp.int32)
        basev = jnp.full((LANE,), base, jnp.int32)

        def gath(buf):
            pltpu.async_copy(ap.at[dsti.at[buf]], bufa.at[buf], sga)
            pltpu.async_copy(bp.at[srci.at[buf]], bufb.at[buf], sgb)

        def wait_idx(buf):
            pltpu.make_async_copy(dsth.at[cid, sid, 0],
                                  dsti.at[buf], sm).wait()
            pltpu.make_async_copy(srch.at[cid, sid, 0],
                                  srci.at[buf], sm).wait()

        def wait_gath(buf):
            pltpu.make_async_copy(ap.at[dsti.at[buf]],
                                  bufa.at[buf], sga).wait()
            pltpu.make_async_copy(bp.at[srci.at[buf]],
                                  bufb.at[buf], sgb).wait()

        gath(0)

        @pl.loop(0, cpw, step=2)
        def _(j):
            for t in range(2):
                jj = j + t
                cur = t
                nxt = 1 - t

                @pl.when(jj + 1 < cpw)
                def _():
                    wait_idx(nxt)
                    gath(nxt)

                wait_gath(cur)

                for f in range(K // LANE):
                    dstl[cur, pl.ds(f * LANE, LANE)] = (
                        dsti[cur, pl.ds(f * LANE, LANE)] - basev)

                @pl.loop(0, K)
                def _(e):
                    s = []
                    acc1 = jnp.zeros((LANE,), jnp.float32)
                    acc2 = jnp.zeros((LANE,), jnp.float32)
                    for f in range(NF):
                        sf = (bufa[cur, e, pl.ds(f * LANE, LANE)]
                              + bufb[cur, e, pl.ds(f * LANE, LANE)])
                        s.append(sf)
                        acc1 = acc1 + sf
                        acc2 = acc2 + sf * sf
                    mu = jnp.sum(acc1) * (1.0 / H)
                    ms = jnp.sum(acc2) * (1.0 / H)
                    var = ms - mu * mu + LN_EPS
                    vv = jnp.full((LANE,), var, jnp.float32)
                    yi = magic - lax.shift_right_logical(
                        plsc.bitcast(vv, jnp.int32), 1)
                    y = plsc.bitcast(yi, jnp.float32)
                    xh = vv * 0.5
                    for _ in range(3):
                        y = y * (1.5 - xh * y * y)
                    muv = jnp.full((LANE,), mu, jnp.float32)
                    for f in range(NF):
                        z = (s[f] - muv) * y * gvec[f] + bvec[f]
                        bufa[cur, e, pl.ds(f * LANE, LANE)] = (
                            jnp.maximum(z, 0.0))

                pltpu.sync_copy(bufa.at[cur], s_sh.at[dstl.at[cur]],
                                add=True)

                @pl.when(jj + 2 < cpw)
                def _():
                    pltpu.async_copy(dsth.at[cid, sid, jj + 2],
                                     dsti.at[cur], sm)
                    pltpu.async_copy(srch.at[cid, sid, jj + 2],
                                     srci.at[cur], sm)

        plsc.subcore_barrier()
        pltpu.sync_copy(s_sh.at[pl.ds(sid * RPT, RPT)],
                        out.at[cid, pl.ds(sid * RPT, RPT)])

    return kern


def _make_sc_hist(cpw):
    mesh = plsc.VectorSubcoreMesh(core_axis_name="c", subcore_axis_name="s")

    @functools.partial(
        pl.kernel,
        out_type=jax.ShapeDtypeStruct((NCORES, TPC, H), jnp.float32),
        mesh=mesh,
        compiler_params=_sc_cp(),
        scratch_types=[
            pltpu.VMEM_SHARED((TPC, H), jnp.float32),
            pltpu.VMEM((K, H), jnp.float32),
            pltpu.VMEM((cpw, K), jnp.int32),
            pltpu.VMEM((LANE,), jnp.int32),
            pltpu.SemaphoreType.DMA,
        ],
    )
    def kern(dsth, zz, ones_rows, out, s_sh, obuf, dstv, _unused, sm):
        cid = lax.axis_index("c")
        sid = lax.axis_index("s")
        base = cid * SPLIT
        basev = jnp.full((LANE,), base, jnp.int32)

        pltpu.async_copy(dsth.at[cid, sid], dstv, sm).wait()
        pltpu.async_copy(ones_rows, obuf, sm).wait()
        pltpu.async_copy(zz.at[pl.ds(sid * RPT, RPT)],
                         s_sh.at[pl.ds(sid * RPT, RPT)], sm).wait()

        @pl.loop(0, cpw)
        def _(jj):
            for f in range(K // LANE):
                dstv[jj, pl.ds(f * LANE, LANE)] = (
                    dstv[jj, pl.ds(f * LANE, LANE)] - basev)

        plsc.subcore_barrier()

        @pl.loop(0, cpw)
        def _(jj):
            pltpu.sync_copy(obuf, s_sh.at[dstv.at[jj]], add=True)

        plsc.subcore_barrier()
        pltpu.sync_copy(s_sh.at[pl.ds(sid * RPT, RPT)],
                        out.at[cid, pl.ds(sid * RPT, RPT)])

    return kern


_SC_EDGE = {cpw: _make_sc_edge(cpw) for cpw in (82, 42)}
_SC_HIST = {cpw: _make_sc_hist(cpw) for cpw in (82, 42)}


def _edge_blocks(edges, cpw):
    e = edges.shape[1]
    capc = NSUB * cpw * K
    src = edges[0].astype(jnp.int32)
    dst = edges[1].astype(jnp.int32)
    side = (dst >= SPLIT).astype(jnp.int32)
    pos0 = jnp.cumsum(1 - side) - 1
    pos1 = jnp.cumsum(side) - 1 + capc
    pos = jnp.where(side == 1, pos1, pos0)
    dfill = jnp.concatenate([jnp.full((capc,), SPLIT, jnp.int32),
                             jnp.full((capc,), SPLIT + SPLIT, jnp.int32)])
    dstb = dfill.at[pos].set(dst).reshape(NCORES, NSUB, cpw, K)
    sfill = jnp.full((2 * capc,), N, jnp.int32)
    srcb = sfill.at[pos].set(src).reshape(NCORES, NSUB, cpw, K)
    return srcb, dstb


def _row(v):
    return v.reshape(1, -1)


def _gw_layer(p, h, srcb, dstb, cnt, zz, cpw):
    msg, upd = p["msg"], p["upd"]
    W1 = msg["l1"]["W"]
    A, B, Sself = _tc_pre(h, W1[:H], W1[H:], _row(msg["l1"]["b"]),
                          _row(msg["ln"]["g"]), _row(msg["ln"]["b"]))
    Ap = jnp.pad(A, ((0, NPAD - N), (0, 0)))
    Bp = jnp.pad(B, ((0, NPAD - N), (0, 0)))
    gb = jnp.stack([msg["ln"]["g"], msg["ln"]["b"]])
    S2 = _SC_EDGE[cpw](Ap, Bp, dstb, srcb, zz, gb)
    S = jnp.concatenate([S2[0, :SPLIT], S2[1, :SPLIT]], axis=0)
    U1 = upd["l1"]["W"]
    return _tc_post(S, cnt, Sself, h,
                    msg["l2"]["W"], _row(msg["l2"]["b"]),
                    U1[:H], U1[H:], _row(upd["l1"]["b"]),
                    _row(upd["ln"]["g"]), _row(upd["ln"]["b"]),
                    upd["l2"]["W"], _row(upd["l2"]["b"]))


def kernel(x, fine_edges, coarse_edges, params):
    srcf, dstf = _edge_blocks(fine_edges, 82)
    srcc, dstc = _edge_blocks(coarse_edges, 42)
    zz = jnp.zeros((TPC, SW), jnp.float32)
    ones_rows = jnp.zeros((K, H), jnp.float32).at[:, 0].set(1.0)

    hist_f = _SC_HIST[82](dstf, zz, ones_rows)
    hist_c = _SC_HIST[42](dstc, zz, ones_rows)
    cf = jnp.concatenate([hist_f[0, :SPLIT, 0:1], hist_f[1, :SPLIT, 0:1]],
                         axis=0)
    cc = jnp.concatenate([hist_c[0, :SPLIT, 0:1], hist_c[1, :SPLIT, 0:1]],
                         axis=0)

    enc = params["enc"]
    h = _tc_enc(x, enc["l1"]["W"], _row(enc["l1"]["b"]),
                _row(enc["ln"]["g"]), _row(enc["ln"]["b"]),
                enc["l2"]["W"], _row(enc["l2"]["b"]))

    hf = h
    for lp in params["fine"]:
        hf = _gw_layer(lp, hf, srcf, dstf, cf, zz, 82)
    hc = h
    for lp in params["coarse"]:
        hc = _gw_layer(lp, hc, srcc, dstc, cc, zz, 42)

    Wm = params["mesh"]["W"]
    return _tc_dec(hf, hc, Wm[:H], Wm[H:], _row(params["mesh"]["b"]),
                   params["dec_l1"]["W"], _row(params["dec_l1"]["b"]),
                   params["dec_l2"]["W"], _row(params["dec_l2"]["b"]))


# speedup vs baseline: 1.8914x; 1.0389x over previous
import dataclasses
import functools

import jax
import jax.numpy as jnp
from jax import lax
from jax.experimental import pallas as pl
from jax.experimental.pallas import tpu as pltpu
from jax.experimental.pallas import tpu_sc as plsc

N = 10000
C = 128
H = 128
FH = 6
LN_EPS = 1e-5

BLK = 2000
NPAD = 10112
SW = 128
K = 128
NCORES = 2
NSUB = 16
LANE = 16
NF = H // LANE
SPLIT = N // 2
TPC = 5120
RPT = TPC // NSUB


def _ln(t, g, b):
    mu = jnp.mean(t, axis=-1, keepdims=True)
    var = jnp.mean((t - mu) ** 2, axis=-1, keepdims=True)
    return (t - mu) * lax.rsqrt(var + LN_EPS) * g + b


def _rows(i):
    return (i, 0)


def _const(i):
    return (0, 0)


def _mm(a, b):
    return jnp.dot(a, b, preferred_element_type=jnp.float32)


def _enc_body(x_ref, w1_ref, b1_ref, g_ref, bb_ref, w2_ref, c2_ref, o_ref):
    t = _mm(x_ref[:], w1_ref[:]) + b1_ref[:]
    y = jnp.maximum(_ln(t, g_ref[:], bb_ref[:]), 0.0)
    o_ref[:] = _mm(y, w2_ref[:]) + c2_ref[:]


def _tc_enc(x, w1, b1, g, bb, w2, c2):
    return pl.pallas_call(
        _enc_body,
        grid=(N // BLK,),
        in_specs=[
            pl.BlockSpec((BLK, C), _rows),
            pl.BlockSpec((C, H), _const),
            pl.BlockSpec((1, H), _const),
            pl.BlockSpec((1, H), _const),
            pl.BlockSpec((1, H), _const),
            pl.BlockSpec((H, H), _const),
            pl.BlockSpec((1, H), _const),
        ],
        out_specs=pl.BlockSpec((BLK, H), _rows),
        out_shape=jax.ShapeDtypeStruct((N, H), jnp.float32),
    )(x, w1, b1, g, bb, w2, c2)


def _pre_body(h_ref, wa_ref, wb_ref, b1_ref, g_ref, bb_ref,
              a_ref, b_ref, s_ref):
    h = h_ref[:]
    A = _mm(h, wa_ref[:]) + b1_ref[:]
    B = _mm(h, wb_ref[:])
    a_ref[:] = A
    b_ref[:] = B
    s_ref[:] = jnp.maximum(_ln(A + B, g_ref[:], bb_ref[:]), 0.0)


def _tc_pre(h, wa, wb, b1, g, bb):
    return pl.pallas_call(
        _pre_body,
        grid=(N // BLK,),
        in_specs=[
            pl.BlockSpec((BLK, H), _rows),
            pl.BlockSpec((H, H), _const),
            pl.BlockSpec((H, H), _const),
            pl.BlockSpec((1, H), _const),
            pl.BlockSpec((1, H), _const),
            pl.BlockSpec((1, H), _const),
        ],
        out_specs=[
            pl.BlockSpec((BLK, H), _rows),
            pl.BlockSpec((BLK, H), _rows),
            pl.BlockSpec((BLK, H), _rows),
        ],
        out_shape=[
            jax.ShapeDtypeStruct((N, H), jnp.float32),
            jax.ShapeDtypeStruct((N, H), jnp.float32),
            jax.ShapeDtypeStruct((N, H), jnp.float32),
        ],
    )(h, wa, wb, b1, g, bb)


def _post_body(s_ref, c_ref, ss_ref, h_ref, w2_ref,
               b2_ref, ua_ref, ub_ref, c1_ref, gu_ref, bu_ref, u2_ref,
               c2_ref, o_ref):
    S = s_ref[:] + ss_ref[:]
    cnt = c_ref[:] + 1.0
    aggr = _mm(S, w2_ref[:]) + cnt * b2_ref[:]
    t = _mm(h_ref[:], ua_ref[:]) + _mm(aggr, ub_ref[:]) + c1_ref[:]
    y = jnp.maximum(_ln(t, gu_ref[:], bu_ref[:]), 0.0)
    o_ref[:] = _mm(y, u2_ref[:]) + c2_ref[:]


def _tc_post(s, c, ss, h, w2, b2, ua, ub, c1, gu, bu, u2, c2):
    return pl.pallas_call(
        _post_body,
        grid=(N // BLK,),
        in_specs=[
            pl.BlockSpec((BLK, H), _rows),
            pl.BlockSpec((BLK, 1), _rows),
            pl.BlockSpec((BLK, H), _rows),
            pl.BlockSpec((BLK, H), _rows),
            pl.BlockSpec((H, H), _const),
            pl.BlockSpec((1, H), _const),
            pl.BlockSpec((H, H), _const),
            pl.BlockSpec((H, H), _const),
            pl.BlockSpec((1, H), _const),
            pl.BlockSpec((1, H), _const),
            pl.BlockSpec((1, H), _const),
            pl.BlockSpec((H, H), _const),
            pl.BlockSpec((1, H), _const),
        ],
        out_specs=pl.BlockSpec((BLK, H), _rows),
        out_shape=jax.ShapeDtypeStruct((N, H), jnp.float32),
    )(s, c, ss, h, w2, b2, ua, ub, c1, gu, bu, u2, c2)


def _dec_body(hf_ref, hc_ref, wma_ref, wmb_ref, bm_ref,
              wd1_ref, bd1_ref, wd2_ref, bd2_ref, o_ref):
    comb = _mm(hf_ref[:], wma_ref[:]) + _mm(hc_ref[:], wmb_ref[:]) + bm_ref[:]
    d = jnp.maximum(_mm(comb, wd1_ref[:]) + bd1_ref[:], 0.0)
    o_ref[:] = _mm(d, wd2_ref[:]) + bd2_ref[:]


def _tc_dec(hf, hc, wma, wmb, bm, wd1, bd1, wd2, bd2):
    OD = FH * C
    return pl.pallas_call(
        _dec_body,
        grid=(N // BLK,),
        in_specs=[
            pl.BlockSpec((BLK, H), _rows),
            pl.BlockSpec((BLK, H), _rows),
            pl.BlockSpec((H, H), _const),
            pl.BlockSpec((H, H), _const),
            pl.BlockSpec((1, H), _const),
            pl.BlockSpec((H, H), _const),
            pl.BlockSpec((1, H), _const),
            pl.BlockSpec((H, OD), _const),
            pl.BlockSpec((1, OD), _const),
        ],
        out_specs=pl.BlockSpec((BLK, OD), _rows),
        out_shape=jax.ShapeDtypeStruct((N, OD), jnp.float32),
    )(hf, hc, wma, wmb, bm, wd1, bd1, wd2, bd2)


def _sc_cp():
    cp = pltpu.CompilerParams()
    if "needs_layout_passes" in pltpu.CompilerParams.__dataclass_fields__:
        cp = dataclasses.replace(cp, needs_layout_passes=False)
    return cp


def _make_sc_edge(cpw):
    mesh = plsc.VectorSubcoreMesh(core_axis_name="c", subcore_axis_name="s")

    @functools.partial(
        pl.kernel,
        out_type=jax.ShapeDtypeStruct((NCORES, TPC, SW), jnp.float32),
        mesh=mesh,
        compiler_params=_sc_cp(),
        scratch_types=[
            pltpu.VMEM_SHARED((TPC, SW), jnp.float32),
            pltpu.VMEM((2, K, H), jnp.float32),
            pltpu.VMEM((2, K, H), jnp.float32),
            pltpu.VMEM((2, K), jnp.int32),
            pltpu.VMEM((2, K), jnp.int32),
            pltpu.VMEM((2, K), jnp.int32),
            pltpu.VMEM((2, H), jnp.float32),
            pltpu.SemaphoreType.DMA,
            pltpu.SemaphoreType.DMA,
            pltpu.SemaphoreType.DMA,
            pltpu.SemaphoreType.DMA,
        ],
    )
    def kern(ap, bp, dsth, srch, zz, gb, out, s_sh, bufa, bufb,
             dsti, srci, dstl, gbv, sga, sgb, sm, ssc):
        cid = lax.axis_index("c")
        sid = lax.axis_index("s")
        base = cid * SPLIT

        pltpu.async_copy(gb, gbv, sm).wait()
        pltpu.async_copy(dsth.at[cid, sid, 0], dsti.at[0], sm).wait()
        pltpu.async_copy(srch.at[cid, sid, 0], srci.at[0], sm).wait()
        pltpu.async_copy(dsth.at[cid, sid, 1], dsti.at[1], sm)
        pltpu.async_copy(srch.at[cid, sid, 1], srci.at[1], sm)
        pltpu.async_copy(zz.at[pl.ds(sid * RPT, RPT)],
                         s_sh.at[pl.ds(sid * RPT, RPT)], sm).wait()
        plsc.subcore_barrier()

        gvec = [gbv[0, pl.ds(f * LANE, LANE)] for f in range(NF)]
        bvec = [gbv[1, pl.ds(f * LANE, LANE)] for f in range(NF)]
        magic = jnp.full((LANE,), 0x5F3759DF, jnp.int32)
        basev = jnp.full((LANE,), base, jnp.int32)

        def gath(buf):
            pltpu.async_copy(ap.at[dsti.at[buf]], bufa.at[buf], sga)
            pltpu.async_copy(bp.at[srci.at[buf]], bufb.at[buf], sgb)

        def wait_idx(buf):
            pltpu.make_async_copy(dsth.at[cid, sid, 0],
                                  dsti.at[buf], sm).wait()
            pltpu.make_async_copy(srch.at[cid, sid, 0],
                                  srci.at[buf], sm).wait()

        def wait_gath(buf):
            pltpu.make_async_copy(ap.at[dsti.at[buf]],
                                  bufa.at[buf], sga).wait()
            pltpu.make_async_copy(bp.at[srci.at[buf]],
                                  bufb.at[buf], sgb).wait()

        def wait_scat(buf):
            pltpu.make_async_copy(bufa.at[buf], s_sh.at[dstl.at[buf]],
                                  ssc).wait()

        def ln_edge(cur, e):
            s = []
            acc1 = jnp.zeros((LANE,), jnp.float32)
            acc2 = jnp.zeros((LANE,), jnp.float32)
            for f in range(NF):
                sf = (bufa[cur, e, pl.ds(f * LANE, LANE)]
                      + bufb[cur, e, pl.ds(f * LANE, LANE)])
                s.append(sf)
                acc1 = acc1 + sf
                acc2 = acc2 + sf * sf
            mu = jnp.sum(acc1) * (1.0 / H)
            ms = jnp.sum(acc2) * (1.0 / H)
            var = ms - mu * mu + LN_EPS
            vv = jnp.full((LANE,), var, jnp.float32)
            yi = magic - lax.shift_right_logical(
                plsc.bitcast(vv, jnp.int32), 1)
            y = plsc.bitcast(yi, jnp.float32)
            xh = vv * 0.5
            for _ in range(3):
                y = y * (1.5 - xh * y * y)
            muv = jnp.full((LANE,), mu, jnp.float32)
            for f in range(NF):
                z = (s[f] - muv) * y * gvec[f] + bvec[f]
                bufa[cur, e, pl.ds(f * LANE, LANE)] = jnp.maximum(z, 0.0)

        gath(0)

        @pl.loop(0, cpw, step=2)
        def _(j):
            for t in range(2):
                jj = j + t
                cur = t
                nxt = 1 - t

                @pl.when(jj >= 1)
                def _():
                    wait_scat(nxt)

                @pl.when(jj + 1 < cpw)
                def _():
                    wait_idx(nxt)
                    gath(nxt)

                wait_gath(cur)

                for f in range(K // LANE):
                    dstl[cur, pl.ds(f * LANE, LANE)] = (
                        dsti[cur, pl.ds(f * LANE, LANE)] - basev)

                @pl.loop(0, K, step=4)
                def _(e0):
                    for u in range(4):
                        ln_edge(cur, e0 + u)

                pltpu.async_copy(bufa.at[cur], s_sh.at[dstl.at[cur]],
                                 ssc, add=True)

                @pl.when(jj + 2 < cpw)
                def _():
                    pltpu.async_copy(dsth.at[cid, sid, jj + 2],
                                     dsti.at[cur], sm)
                    pltpu.async_copy(srch.at[cid, sid, jj + 2],
                                     srci.at[cur], sm)

        wait_scat(1)
        plsc.subcore_barrier()
        pltpu.sync_copy(s_sh.at[pl.ds(sid * RPT, RPT)],
                        out.at[cid, pl.ds(sid * RPT, RPT)])

    return kern


def _make_sc_hist(cpw):
    mesh = plsc.VectorSubcoreMesh(core_axis_name="c", subcore_axis_name="s")

    @functools.partial(
        pl.kernel,
        out_type=jax.ShapeDtypeStruct((NCORES, TPC, H), jnp.float32),
        mesh=mesh,
        compiler_params=_sc_cp(),
        scratch_types=[
            pltpu.VMEM_SHARED((TPC, H), jnp.float32),
            pltpu.VMEM((K, H), jnp.float32),
            pltpu.VMEM((cpw, K), jnp.int32),
            pltpu.VMEM((LANE,), jnp.int32),
            pltpu.SemaphoreType.DMA,
        ],
    )
    def kern(dsth, zz, ones_rows, out, s_sh, obuf, dstv, _unused, sm):
        cid = lax.axis_index("c")
        sid = lax.axis_index("s")
        base = cid * SPLIT
        basev = jnp.full((LANE,), base, jnp.int32)

        pltpu.async_copy(dsth.at[cid, sid], dstv, sm).wait()
        pltpu.async_copy(ones_rows, obuf, sm).wait()
        pltpu.async_copy(zz.at[pl.ds(sid * RPT, RPT)],
                         s_sh.at[pl.ds(sid * RPT, RPT)], sm).wait()

        @pl.loop(0, cpw)
        def _(jj):
            for f in range(K // LANE):
                dstv[jj, pl.ds(f * LANE, LANE)] = (
                    dstv[jj, pl.ds(f * LANE, LANE)] - basev)

        plsc.subcore_barrier()

        @pl.loop(0, cpw)
        def _(jj):
            pltpu.sync_copy(obuf, s_sh.at[dstv.at[jj]], add=True)

        plsc.subcore_barrier()
        pltpu.sync_copy(s_sh.at[pl.ds(sid * RPT, RPT)],
                        out.at[cid, pl.ds(sid * RPT, RPT)])

    return kern


_SC_EDGE = {cpw: _make_sc_edge(cpw) for cpw in (82, 42)}
_SC_HIST = {cpw: _make_sc_hist(cpw) for cpw in (82, 42)}


def _edge_blocks(edges, cpw):
    e = edges.shape[1]
    capc = NSUB * cpw * K
    src = edges[0].astype(jnp.int32)
    dst = edges[1].astype(jnp.int32)
    side = (dst >= SPLIT).astype(jnp.int32)
    pos0 = jnp.cumsum(1 - side) - 1
    pos1 = jnp.cumsum(side) - 1 + capc
    pos = jnp.where(side == 1, pos1, pos0)
    dfill = jnp.concatenate([jnp.full((capc,), SPLIT, jnp.int32),
                             jnp.full((capc,), SPLIT + SPLIT, jnp.int32)])
    dstb = dfill.at[pos].set(dst).reshape(NCORES, NSUB, cpw, K)
    sfill = jnp.full((2 * capc,), N, jnp.int32)
    srcb = sfill.at[pos].set(src).reshape(NCORES, NSUB, cpw, K)
    return srcb, dstb


def _row(v):
    return v.reshape(1, -1)


def _gw_layer(p, h, srcb, dstb, cnt, zz, cpw):
    msg, upd = p["msg"], p["upd"]
    W1 = msg["l1"]["W"]
    A, B, Sself = _tc_pre(h, W1[:H], W1[H:], _row(msg["l1"]["b"]),
                          _row(msg["ln"]["g"]), _row(msg["ln"]["b"]))
    Ap = jnp.pad(A, ((0, NPAD - N), (0, 0)))
    Bp = jnp.pad(B, ((0, NPAD - N), (0, 0)))
    gb = jnp.stack([msg["ln"]["g"], msg["ln"]["b"]])
    S2 = _SC_EDGE[cpw](Ap, Bp, dstb, srcb, zz, gb)
    S = jnp.concatenate([S2[0, :SPLIT], S2[1, :SPLIT]], axis=0)
    U1 = upd["l1"]["W"]
    return _tc_post(S, cnt, Sself, h,
                    msg["l2"]["W"], _row(msg["l2"]["b"]),
                    U1[:H], U1[H:], _row(upd["l1"]["b"]),
                    _row(upd["ln"]["g"]), _row(upd["ln"]["b"]),
                    upd["l2"]["W"], _row(upd["l2"]["b"]))


def kernel(x, fine_edges, coarse_edges, params):
    srcf, dstf = _edge_blocks(fine_edges, 82)
    srcc, dstc = _edge_blocks(coarse_edges, 42)
    zz = jnp.zeros((TPC, SW), jnp.float32)
    ones_rows = jnp.zeros((K, H), jnp.float32).at[:, 0].set(1.0)

    hist_f = _SC_HIST[82](dstf, zz, ones_rows)
    hist_c = _SC_HIST[42](dstc, zz, ones_rows)
    cf = jnp.concatenate([hist_f[0, :SPLIT, 0:1], hist_f[1, :SPLIT, 0:1]],
                         axis=0)
    cc = jnp.concatenate([hist_c[0, :SPLIT, 0:1], hist_c[1, :SPLIT, 0:1]],
                         axis=0)

    enc = params["enc"]
    h = _tc_enc(x, enc["l1"]["W"], _row(enc["l1"]["b"]),
                _row(enc["ln"]["g"]), _row(enc["ln"]["b"]),
                enc["l2"]["W"], _row(enc["l2"]["b"]))

    hf = h
    for lp in params["fine"]:
        hf = _gw_layer(lp, hf, srcf, dstf, cf, zz, 82)
    hc = h
    for lp in params["coarse"]:
        hc = _gw_layer(lp, hc, srcc, dstc, cc, zz, 42)

    Wm = params["mesh"]["W"]
    return _tc_dec(hf, hc, Wm[:H], Wm[H:], _row(params["mesh"]["b"]),
                   params["dec_l1"]["W"], _row(params["dec_l1"]["b"]),
                   params["dec_l2"]["W"], _row(params["dec_l2"]["b"]))


# speedup vs baseline: 1.9015x; 1.0053x over previous
import dataclasses
import functools

import jax
import jax.numpy as jnp
from jax import lax
from jax.experimental import pallas as pl
from jax.experimental.pallas import tpu as pltpu
from jax.experimental.pallas import tpu_sc as plsc

N = 10000
C = 128
H = 128
FH = 6
LN_EPS = 1e-5

BLK = 2000
NPAD = 10112
SW = 128
K = 128
NCORES = 2
NSUB = 16
LANE = 16
NF = H // LANE
SPLIT = N // 2
TPC = 5120
RPT = TPC // NSUB


def _ln(t, g, b):
    mu = jnp.mean(t, axis=-1, keepdims=True)
    var = jnp.mean((t - mu) ** 2, axis=-1, keepdims=True)
    return (t - mu) * lax.rsqrt(var + LN_EPS) * g + b


def _rows(i):
    return (i, 0)


def _const(i):
    return (0, 0)


def _mm(a, b):
    return jnp.dot(a, b, preferred_element_type=jnp.float32)


def _enc_body(x_ref, w1_ref, b1_ref, g_ref, bb_ref, w2_ref, c2_ref, o_ref):
    t = _mm(x_ref[:], w1_ref[:]) + b1_ref[:]
    y = jnp.maximum(_ln(t, g_ref[:], bb_ref[:]), 0.0)
    o_ref[:] = _mm(y, w2_ref[:]) + c2_ref[:]


def _tc_enc(x, w1, b1, g, bb, w2, c2):
    return pl.pallas_call(
        _enc_body,
        grid=(N // BLK,),
        in_specs=[
            pl.BlockSpec((BLK, C), _rows),
            pl.BlockSpec((C, H), _const),
            pl.BlockSpec((1, H), _const),
            pl.BlockSpec((1, H), _const),
            pl.BlockSpec((1, H), _const),
            pl.BlockSpec((H, H), _const),
            pl.BlockSpec((1, H), _const),
        ],
        out_specs=pl.BlockSpec((BLK, H), _rows),
        out_shape=jax.ShapeDtypeStruct((N, H), jnp.float32),
    )(x, w1, b1, g, bb, w2, c2)


def _pre_body(h_ref, wa_ref, wb_ref, b1_ref, g_ref, bb_ref,
              a_ref, b_ref, s_ref):
    h = h_ref[:]
    A = _mm(h, wa_ref[:]) + b1_ref[:]
    B = _mm(h, wb_ref[:])
    a_ref[:] = A
    b_ref[:] = B
    s_ref[:] = jnp.maximum(_ln(A + B, g_ref[:], bb_ref[:]), 0.0)


def _tc_pre(h, wa, wb, b1, g, bb):
    return pl.pallas_call(
        _pre_body,
        grid=(N // BLK,),
        in_specs=[
            pl.BlockSpec((BLK, H), _rows),
            pl.BlockSpec((H, H), _const),
            pl.BlockSpec((H, H), _const),
            pl.BlockSpec((1, H), _const),
            pl.BlockSpec((1, H), _const),
            pl.BlockSpec((1, H), _const),
        ],
        out_specs=[
            pl.BlockSpec((BLK, H), _rows),
            pl.BlockSpec((BLK, H), _rows),
            pl.BlockSpec((BLK, H), _rows),
        ],
        out_shape=[
            jax.ShapeDtypeStruct((N, H), jnp.float32),
            jax.ShapeDtypeStruct((N, H), jnp.float32),
            jax.ShapeDtypeStruct((N, H), jnp.float32),
        ],
    )(h, wa, wb, b1, g, bb)


def _post_body(s_ref, c_ref, ss_ref, h_ref, w2_ref,
               b2_ref, ua_ref, ub_ref, c1_ref, gu_ref, bu_ref, u2_ref,
               c2_ref, o_ref):
    S = s_ref[:] + ss_ref[:]
    cnt = c_ref[:] + 1.0
    aggr = _mm(S, w2_ref[:]) + cnt * b2_ref[:]
    t = _mm(h_ref[:], ua_ref[:]) + _mm(aggr, ub_ref[:]) + c1_ref[:]
    y = jnp.maximum(_ln(t, gu_ref[:], bu_ref[:]), 0.0)
    o_ref[:] = _mm(y, u2_ref[:]) + c2_ref[:]


def _tc_post(s, c, ss, h, w2, b2, ua, ub, c1, gu, bu, u2, c2):
    return pl.pallas_call(
        _post_body,
        grid=(N // BLK,),
        in_specs=[
            pl.BlockSpec((BLK, H), _rows),
            pl.BlockSpec((BLK, 1), _rows),
            pl.BlockSpec((BLK, H), _rows),
            pl.BlockSpec((BLK, H), _rows),
            pl.BlockSpec((H, H), _const),
            pl.BlockSpec((1, H), _const),
            pl.BlockSpec((H, H), _const),
            pl.BlockSpec((H, H), _const),
            pl.BlockSpec((1, H), _const),
            pl.BlockSpec((1, H), _const),
            pl.BlockSpec((1, H), _const),
            pl.BlockSpec((H, H), _const),
            pl.BlockSpec((1, H), _const),
        ],
        out_specs=pl.BlockSpec((BLK, H), _rows),
        out_shape=jax.ShapeDtypeStruct((N, H), jnp.float32),
    )(s, c, ss, h, w2, b2, ua, ub, c1, gu, bu, u2, c2)


def _dec_body(hf_ref, hc_ref, wma_ref, wmb_ref, bm_ref,
              wd1_ref, bd1_ref, wd2_ref, bd2_ref, o_ref):
    comb = _mm(hf_ref[:], wma_ref[:]) + _mm(hc_ref[:], wmb_ref[:]) + bm_ref[:]
    d = jnp.maximum(_mm(comb, wd1_ref[:]) + bd1_ref[:], 0.0)
    o_ref[:] = _mm(d, wd2_ref[:]) + bd2_ref[:]


def _tc_dec(hf, hc, wma, wmb, bm, wd1, bd1, wd2, bd2):
    OD = FH * C
    return pl.pallas_call(
        _dec_body,
        grid=(N // BLK,),
        in_specs=[
            pl.BlockSpec((BLK, H), _rows),
            pl.BlockSpec((BLK, H), _rows),
            pl.BlockSpec((H, H), _const),
            pl.BlockSpec((H, H), _const),
            pl.BlockSpec((1, H), _const),
            pl.BlockSpec((H, H), _const),
            pl.BlockSpec((1, H), _const),
            pl.BlockSpec((H, OD), _const),
            pl.BlockSpec((1, OD), _const),
        ],
        out_specs=pl.BlockSpec((BLK, OD), _rows),
        out_shape=jax.ShapeDtypeStruct((N, OD), jnp.float32),
    )(hf, hc, wma, wmb, bm, wd1, bd1, wd2, bd2)


def _sc_cp():
    cp = pltpu.CompilerParams()
    if "needs_layout_passes" in pltpu.CompilerParams.__dataclass_fields__:
        cp = dataclasses.replace(cp, needs_layout_passes=False)
    return cp


def _make_sc_edge(cpw):
    mesh = plsc.VectorSubcoreMesh(core_axis_name="c", subcore_axis_name="s")

    @functools.partial(
        pl.kernel,
        out_type=jax.ShapeDtypeStruct((NCORES, TPC, SW), jnp.float32),
        mesh=mesh,
        compiler_params=_sc_cp(),
        scratch_types=[
            pltpu.VMEM_SHARED((TPC, SW), jnp.float32),
            pltpu.VMEM((2, K, H), jnp.float32),
            pltpu.VMEM((2, K, H), jnp.float32),
            pltpu.VMEM((2, K), jnp.int32),
            pltpu.VMEM((2, K), jnp.int32),
            pltpu.VMEM((2, K), jnp.int32),
            pltpu.VMEM((2, H), jnp.float32),
            pltpu.VMEM((K,), jnp.float32),
            pltpu.VMEM((K,), jnp.float32),
            pltpu.SemaphoreType.DMA,
            pltpu.SemaphoreType.DMA,
            pltpu.SemaphoreType.DMA,
            pltpu.SemaphoreType.DMA,
        ],
    )
    def kern(ap, bp, dsth, srch, zz, gb, out, s_sh, bufa, bufb,
             dsti, srci, dstl, gbv, st1, st2, sga, sgb, sm, ssc):
        cid = lax.axis_index("c")
        sid = lax.axis_index("s")
        base = cid * SPLIT

        pltpu.async_copy(gb, gbv, sm).wait()
        pltpu.async_copy(dsth.at[cid, sid, 0], dsti.at[0], sm).wait()
        pltpu.async_copy(srch.at[cid, sid, 0], srci.at[0], sm).wait()
        pltpu.async_copy(dsth.at[cid, sid, 1], dsti.at[1], sm)
        pltpu.async_copy(srch.at[cid, sid, 1], srci.at[1], sm)
        pltpu.async_copy(zz.at[pl.ds(sid * RPT, RPT)],
                         s_sh.at[pl.ds(sid * RPT, RPT)], sm).wait()
        plsc.subcore_barrier()

        gvec = [gbv[0, pl.ds(f * LANE, LANE)] for f in range(NF)]
        bvec = [gbv[1, pl.ds(f * LANE, LANE)] for f in range(NF)]
        magic = jnp.full((LANE,), 0x5F3759DF, jnp.int32)
        basev = jnp.full((LANE,), base, jnp.int32)
        m15 = lax.iota(jnp.int32, LANE) == (LANE - 1)

        def gath(buf):
            pltpu.async_copy(ap.at[dsti.at[buf]], bufa.at[buf], sga)
            pltpu.async_copy(bp.at[srci.at[buf]], bufb.at[buf], sgb)

        def wait_idx(buf):
            pltpu.make_async_copy(dsth.at[cid, sid, 0],
                                  dsti.at[buf], sm).wait()
            pltpu.make_async_copy(srch.at[cid, sid, 0],
                                  srci.at[buf], sm).wait()

        def wait_gath(buf):
            pltpu.make_async_copy(ap.at[dsti.at[buf]],
                                  bufa.at[buf], sga).wait()
            pltpu.make_async_copy(bp.at[srci.at[buf]],
                                  bufb.at[buf], sgb).wait()

        def wait_scat(buf):
            pltpu.make_async_copy(bufa.at[buf], s_sh.at[dstl.at[buf]],
                                  ssc).wait()

        def sums_edge(cur, e):
            ev = jnp.full((LANE,), e, jnp.int32)
            acc1 = jnp.zeros((LANE,), jnp.float32)
            acc2 = jnp.zeros((LANE,), jnp.float32)
            for f in range(NF):
                sf = (bufa[cur, e, pl.ds(f * LANE, LANE)]
                      + bufb[cur, e, pl.ds(f * LANE, LANE)])
                bufa[cur, e, pl.ds(f * LANE, LANE)] = sf
                acc1 = acc1 + sf
                acc2 = acc2 + sf * sf
            c1 = plsc.cumsum(acc1)
            c2 = plsc.cumsum(acc2)
            plsc.store_scatter(st1, [ev], c1, mask=m15)
            plsc.store_scatter(st2, [ev], c2, mask=m15)

        def stats_group(g):
            mu = st1[pl.ds(g, LANE)] * (1.0 / H)
            ms = st2[pl.ds(g, LANE)] * (1.0 / H)
            var = ms - mu * mu + LN_EPS
            yi = magic - lax.shift_right_logical(
                plsc.bitcast(var, jnp.int32), 1)
            y = plsc.bitcast(yi, jnp.float32)
            xh = var * 0.5
            for _ in range(3):
                y = y * (1.5 - xh * y * y)
            st1[pl.ds(g, LANE)] = mu
            st2[pl.ds(g, LANE)] = y

        def norm_edge(cur, e):
            ev = jnp.full((LANE,), e, jnp.int32)
            muv = plsc.load_gather(st1, [ev])
            yv = plsc.load_gather(st2, [ev])
            for f in range(NF):
                sf = bufa[cur, e, pl.ds(f * LANE, LANE)]
                z = (sf - muv) * yv * gvec[f] + bvec[f]
                bufa[cur, e, pl.ds(f * LANE, LANE)] = jnp.maximum(z, 0.0)

        gath(0)

        @pl.loop(0, cpw, step=2)
        def _(j):
            for t in range(2):
                jj = j + t
                cur = t
                nxt = 1 - t

                @pl.when(jj >= 1)
                def _():
                    wait_scat(nxt)

                @pl.when(jj + 1 < cpw)
                def _():
                    wait_idx(nxt)
                    gath(nxt)

                wait_gath(cur)

                for f in range(K // LANE):
                    dstl[cur, pl.ds(f * LANE, LANE)] = (
                        dsti[cur, pl.ds(f * LANE, LANE)] - basev)

                @plsc.parallel_loop(0, K, unroll=4)
                def _(e):
                    sums_edge(cur, e)

                @plsc.parallel_loop(0, K, step=LANE, unroll=2)
                def _(g):
                    stats_group(g)

                @plsc.parallel_loop(0, K, unroll=4)
                def _(e):
                    norm_edge(cur, e)

                pltpu.async_copy(bufa.at[cur], s_sh.at[dstl.at[cur]],
                                 ssc, add=True)

                @pl.when(jj + 2 < cpw)
                def _():
                    pltpu.async_copy(dsth.at[cid, sid, jj + 2],
                                     dsti.at[cur], sm)
                    pltpu.async_copy(srch.at[cid, sid, jj + 2],
                                     srci.at[cur], sm)

        wait_scat(1)
        plsc.subcore_barrier()
        pltpu.sync_copy(s_sh.at[pl.ds(sid * RPT, RPT)],
                        out.at[cid, pl.ds(sid * RPT, RPT)])

    return kern


def _make_sc_hist(cpw):
    mesh = plsc.VectorSubcoreMesh(core_axis_name="c", subcore_axis_name="s")

    @functools.partial(
        pl.kernel,
        out_type=jax.ShapeDtypeStruct((NCORES, TPC, H), jnp.float32),
        mesh=mesh,
        compiler_params=_sc_cp(),
        scratch_types=[
            pltpu.VMEM_SHARED((TPC, H), jnp.float32),
            pltpu.VMEM((K, H), jnp.float32),
            pltpu.VMEM((cpw, K), jnp.int32),
            pltpu.VMEM((LANE,), jnp.int32),
            pltpu.SemaphoreType.DMA,
        ],
    )
    def kern(dsth, zz, ones_rows, out, s_sh, obuf, dstv, _unused, sm):
        cid = lax.axis_index("c")
        sid = lax.axis_index("s")
        base = cid * SPLIT
        basev = jnp.full((LANE,), base, jnp.int32)

        pltpu.async_copy(dsth.at[cid, sid], dstv, sm).wait()
        pltpu.async_copy(ones_rows, obuf, sm).wait()
        pltpu.async_copy(zz.at[pl.ds(sid * RPT, RPT)],
                         s_sh.at[pl.ds(sid * RPT, RPT)], sm).wait()

        @pl.loop(0, cpw)
        def _(jj):
            for f in range(K // LANE):
                dstv[jj, pl.ds(f * LANE, LANE)] = (
                    dstv[jj, pl.ds(f * LANE, LANE)] - basev)

        plsc.subcore_barrier()

        @pl.loop(0, cpw)
        def _(jj):
            pltpu.sync_copy(obuf, s_sh.at[dstv.at[jj]], add=True)

        plsc.subcore_barrier()
        pltpu.sync_copy(s_sh.at[pl.ds(sid * RPT, RPT)],
                        out.at[cid, pl.ds(sid * RPT, RPT)])

    return kern


_SC_EDGE = {cpw: _make_sc_edge(cpw) for cpw in (82, 42)}
_SC_HIST = {cpw: _make_sc_hist(cpw) for cpw in (82, 42)}


def _edge_blocks(edges, cpw):
    e = edges.shape[1]
    capc = NSUB * cpw * K
    src = edges[0].astype(jnp.int32)
    dst = edges[1].astype(jnp.int32)
    side = (dst >= SPLIT).astype(jnp.int32)
    pos0 = jnp.cumsum(1 - side) - 1
    pos1 = jnp.cumsum(side) - 1 + capc
    pos = jnp.where(side == 1, pos1, pos0)
    dfill = jnp.concatenate([jnp.full((capc,), SPLIT, jnp.int32),
                             jnp.full((capc,), SPLIT + SPLIT, jnp.int32)])
    dstb = dfill.at[pos].set(dst).reshape(NCORES, NSUB, cpw, K)
    sfill = jnp.full((2 * capc,), N, jnp.int32)
    srcb = sfill.at[pos].set(src).reshape(NCORES, NSUB, cpw, K)
    return srcb, dstb


def _row(v):
    return v.reshape(1, -1)


def _gw_layer(p, h, srcb, dstb, cnt, zz, cpw):
    msg, upd = p["msg"], p["upd"]
    W1 = msg["l1"]["W"]
    A, B, Sself = _tc_pre(h, W1[:H], W1[H:], _row(msg["l1"]["b"]),
                          _row(msg["ln"]["g"]), _row(msg["ln"]["b"]))
    Ap = jnp.pad(A, ((0, NPAD - N), (0, 0)))
    Bp = jnp.pad(B, ((0, NPAD - N), (0, 0)))
    gb = jnp.stack([msg["ln"]["g"], msg["ln"]["b"]])
    S2 = _SC_EDGE[cpw](Ap, Bp, dstb, srcb, zz, gb)
    S = jnp.concatenate([S2[0, :SPLIT], S2[1, :SPLIT]], axis=0)
    U1 = upd["l1"]["W"]
    return _tc_post(S, cnt, Sself, h,
                    msg["l2"]["W"], _row(msg["l2"]["b"]),
                    U1[:H], U1[H:], _row(upd["l1"]["b"]),
                    _row(upd["ln"]["g"]), _row(upd["ln"]["b"]),
                    upd["l2"]["W"], _row(upd["l2"]["b"]))


def kernel(x, fine_edges, coarse_edges, params):
    srcf, dstf = _edge_blocks(fine_edges, 82)
    srcc, dstc = _edge_blocks(coarse_edges, 42)
    zz = jnp.zeros((TPC, SW), jnp.float32)
    ones_rows = jnp.zeros((K, H), jnp.float32).at[:, 0].set(1.0)

    hist_f = _SC_HIST[82](dstf, zz, ones_rows)
    hist_c = _SC_HIST[42](dstc, zz, ones_rows)
    cf = jnp.concatenate([hist_f[0, :SPLIT, 0:1], hist_f[1, :SPLIT, 0:1]],
                         axis=0)
    cc = jnp.concatenate([hist_c[0, :SPLIT, 0:1], hist_c[1, :SPLIT, 0:1]],
                         axis=0)

    enc = params["enc"]
    h = _tc_enc(x, enc["l1"]["W"], _row(enc["l1"]["b"]),
                _row(enc["ln"]["g"]), _row(enc["ln"]["b"]),
                enc["l2"]["W"], _row(enc["l2"]["b"]))

    hf = h
    for lp in params["fine"]:
        hf = _gw_layer(lp, hf, srcf, dstf, cf, zz, 82)
    hc = h
    for lp in params["coarse"]:
        hc = _gw_layer(lp, hc, srcc, dstc, cc, zz, 42)

    Wm = params["mesh"]["W"]
    return _tc_dec(hf, hc, Wm[:H], Wm[H:], _row(params["mesh"]["b"]),
                   params["dec_l1"]["W"], _row(params["dec_l1"]["b"]),
                   params["dec_l2"]["W"], _row(params["dec_l2"]["b"]))


# speedup vs baseline: 1.9069x; 1.0028x over previous
import dataclasses
import functools

import jax
import jax.numpy as jnp
from jax import lax
from jax.experimental import pallas as pl
from jax.experimental.pallas import tpu as pltpu
from jax.experimental.pallas import tpu_sc as plsc

N = 10000
C = 128
H = 128
FH = 6
LN_EPS = 1e-5

BLK = 2000
NPAD = 10112
SW = 128
K = 128
NCORES = 2
NSUB = 16
LANE = 16
NF = H // LANE
SPLIT = N // 2
TPC = 5120
RPT = TPC // NSUB


def _ln(t, g, b):
    mu = jnp.mean(t, axis=-1, keepdims=True)
    var = jnp.mean((t - mu) ** 2, axis=-1, keepdims=True)
    return (t - mu) * lax.rsqrt(var + LN_EPS) * g + b


def _rows(i):
    return (i, 0)


def _const(i):
    return (0, 0)


def _mm(a, b):
    return jnp.dot(a, b, preferred_element_type=jnp.float32)


def _enc_body(x_ref, w1_ref, b1_ref, g_ref, bb_ref, w2_ref, c2_ref, o_ref):
    t = _mm(x_ref[:], w1_ref[:]) + b1_ref[:]
    y = jnp.maximum(_ln(t, g_ref[:], bb_ref[:]), 0.0)
    o_ref[:] = _mm(y, w2_ref[:]) + c2_ref[:]


def _tc_enc(x, w1, b1, g, bb, w2, c2):
    return pl.pallas_call(
        _enc_body,
        grid=(N // BLK,),
        in_specs=[
            pl.BlockSpec((BLK, C), _rows),
            pl.BlockSpec((C, H), _const),
            pl.BlockSpec((1, H), _const),
            pl.BlockSpec((1, H), _const),
            pl.BlockSpec((1, H), _const),
            pl.BlockSpec((H, H), _const),
            pl.BlockSpec((1, H), _const),
        ],
        out_specs=pl.BlockSpec((BLK, H), _rows),
        out_shape=jax.ShapeDtypeStruct((N, H), jnp.float32),
    )(x, w1, b1, g, bb, w2, c2)


def _pre_body(h_ref, wa_ref, wb_ref, b1_ref, g_ref, bb_ref,
              a_ref, b_ref, s_ref):
    h = h_ref[:]
    A = _mm(h, wa_ref[:]) + b1_ref[:]
    B = _mm(h, wb_ref[:])
    a_ref[:] = A
    b_ref[:] = B
    s_ref[:] = jnp.maximum(_ln(A + B, g_ref[:], bb_ref[:]), 0.0)


def _tc_pre(h, wa, wb, b1, g, bb):
    return pl.pallas_call(
        _pre_body,
        grid=(N // BLK,),
        in_specs=[
            pl.BlockSpec((BLK, H), _rows),
            pl.BlockSpec((H, H), _const),
            pl.BlockSpec((H, H), _const),
            pl.BlockSpec((1, H), _const),
            pl.BlockSpec((1, H), _const),
            pl.BlockSpec((1, H), _const),
        ],
        out_specs=[
            pl.BlockSpec((BLK, H), _rows),
            pl.BlockSpec((BLK, H), _rows),
            pl.BlockSpec((BLK, H), _rows),
        ],
        out_shape=[
            jax.ShapeDtypeStruct((N, H), jnp.float32),
            jax.ShapeDtypeStruct((N, H), jnp.float32),
            jax.ShapeDtypeStruct((N, H), jnp.float32),
        ],
    )(h, wa, wb, b1, g, bb)


def _post_body(s_ref, c_ref, ss_ref, h_ref, w2_ref,
               b2_ref, ua_ref, ub_ref, c1_ref, gu_ref, bu_ref, u2_ref,
               c2_ref, o_ref):
    S = s_ref[:] + ss_ref[:]
    cnt = c_ref[:] + 1.0
    aggr = _mm(S, w2_ref[:]) + cnt * b2_ref[:]
    t = _mm(h_ref[:], ua_ref[:]) + _mm(aggr, ub_ref[:]) + c1_ref[:]
    y = jnp.maximum(_ln(t, gu_ref[:], bu_ref[:]), 0.0)
    o_ref[:] = _mm(y, u2_ref[:]) + c2_ref[:]


def _tc_post(s, c, ss, h, w2, b2, ua, ub, c1, gu, bu, u2, c2):
    return pl.pallas_call(
        _post_body,
        grid=(N // BLK,),
        in_specs=[
            pl.BlockSpec((BLK, H), _rows),
            pl.BlockSpec((BLK, 1), _rows),
            pl.BlockSpec((BLK, H), _rows),
            pl.BlockSpec((BLK, H), _rows),
            pl.BlockSpec((H, H), _const),
            pl.BlockSpec((1, H), _const),
            pl.BlockSpec((H, H), _const),
            pl.BlockSpec((H, H), _const),
            pl.BlockSpec((1, H), _const),
            pl.BlockSpec((1, H), _const),
            pl.BlockSpec((1, H), _const),
            pl.BlockSpec((H, H), _const),
            pl.BlockSpec((1, H), _const),
        ],
        out_specs=pl.BlockSpec((BLK, H), _rows),
        out_shape=jax.ShapeDtypeStruct((N, H), jnp.float32),
    )(s, c, ss, h, w2, b2, ua, ub, c1, gu, bu, u2, c2)


def _dec_body(hf_ref, hc_ref, wma_ref, wmb_ref, bm_ref,
              wd1_ref, bd1_ref, wd2_ref, bd2_ref, o_ref):
    comb = _mm(hf_ref[:], wma_ref[:]) + _mm(hc_ref[:], wmb_ref[:]) + bm_ref[:]
    d = jnp.maximum(_mm(comb, wd1_ref[:]) + bd1_ref[:], 0.0)
    o_ref[:] = _mm(d, wd2_ref[:]) + bd2_ref[:]


def _tc_dec(hf, hc, wma, wmb, bm, wd1, bd1, wd2, bd2):
    OD = FH * C
    return pl.pallas_call(
        _dec_body,
        grid=(N // BLK,),
        in_specs=[
            pl.BlockSpec((BLK, H), _rows),
            pl.BlockSpec((BLK, H), _rows),
            pl.BlockSpec((H, H), _const),
            pl.BlockSpec((H, H), _const),
            pl.BlockSpec((1, H), _const),
            pl.BlockSpec((H, H), _const),
            pl.BlockSpec((1, H), _const),
            pl.BlockSpec((H, OD), _const),
            pl.BlockSpec((1, OD), _const),
        ],
        out_specs=pl.BlockSpec((BLK, OD), _rows),
        out_shape=jax.ShapeDtypeStruct((N, OD), jnp.float32),
    )(hf, hc, wma, wmb, bm, wd1, bd1, wd2, bd2)


def _sc_cp():
    cp = pltpu.CompilerParams()
    if "needs_layout_passes" in pltpu.CompilerParams.__dataclass_fields__:
        cp = dataclasses.replace(cp, needs_layout_passes=False)
    return cp


def _make_sc_edge(cpw):
    mesh = plsc.VectorSubcoreMesh(core_axis_name="c", subcore_axis_name="s")

    @functools.partial(
        pl.kernel,
        out_type=[
            jax.ShapeDtypeStruct((NCORES, TPC, SW), jnp.float32),
            jax.ShapeDtypeStruct((NCORES, NSUB, TPC), jnp.float32),
        ],
        mesh=mesh,
        compiler_params=_sc_cp(),
        scratch_types=[
            pltpu.VMEM_SHARED((TPC, SW), jnp.float32),
            pltpu.VMEM((2, K, H), jnp.float32),
            pltpu.VMEM((2, K, H), jnp.float32),
            pltpu.VMEM((2, K), jnp.int32),
            pltpu.VMEM((2, K), jnp.int32),
            pltpu.VMEM((2, K), jnp.int32),
            pltpu.VMEM((2, H), jnp.float32),
            pltpu.VMEM((K,), jnp.float32),
            pltpu.VMEM((K,), jnp.float32),
            pltpu.VMEM((TPC,), jnp.float32),
            pltpu.SemaphoreType.DMA,
            pltpu.SemaphoreType.DMA,
            pltpu.SemaphoreType.DMA,
            pltpu.SemaphoreType.DMA,
        ],
    )
    def kern(ap, bp, dsth, srch, zz, gb, out, outc, s_sh, bufa, bufb,
             dsti, srci, dstl, gbv, st1, st2, cntv, sga, sgb, sm, ssc):
        cid = lax.axis_index("c")
        sid = lax.axis_index("s")
        base = cid * SPLIT

        pltpu.async_copy(gb, gbv, sm).wait()
        pltpu.async_copy(dsth.at[cid, sid, 0], dsti.at[0], sm).wait()
        pltpu.async_copy(srch.at[cid, sid, 0], srci.at[0], sm).wait()
        pltpu.async_copy(dsth.at[cid, sid, 1], dsti.at[1], sm)
        pltpu.async_copy(srch.at[cid, sid, 1], srci.at[1], sm)
        pltpu.async_copy(zz.at[pl.ds(sid * RPT, RPT)],
                         s_sh.at[pl.ds(sid * RPT, RPT)], sm).wait()
        plsc.subcore_barrier()

        gvec = [gbv[0, pl.ds(f * LANE, LANE)] for f in range(NF)]
        bvec = [gbv[1, pl.ds(f * LANE, LANE)] for f in range(NF)]
        magic = jnp.full((LANE,), 0x5F3759DF, jnp.int32)
        basev = jnp.full((LANE,), base, jnp.int32)
        m15 = lax.iota(jnp.int32, LANE) == (LANE - 1)
        ones16 = jnp.full((LANE,), 1.0, jnp.float32)
        zeros16 = jnp.zeros((LANE,), jnp.float32)

        @plsc.parallel_loop(0, TPC, step=LANE)
        def _(i):
            cntv[pl.ds(i, LANE)] = zeros16

        def gath(buf):
            pltpu.async_copy(ap.at[dsti.at[buf]], bufa.at[buf], sga)
            pltpu.async_copy(bp.at[srci.at[buf]], bufb.at[buf], sgb)

        def wait_idx(buf):
            pltpu.make_async_copy(dsth.at[cid, sid, 0],
                                  dsti.at[buf], sm).wait()
            pltpu.make_async_copy(srch.at[cid, sid, 0],
                                  srci.at[buf], sm).wait()

        def wait_gath(buf):
            pltpu.make_async_copy(ap.at[dsti.at[buf]],
                                  bufa.at[buf], sga).wait()
            pltpu.make_async_copy(bp.at[srci.at[buf]],
                                  bufb.at[buf], sgb).wait()

        def wait_scat(buf):
            pltpu.make_async_copy(bufa.at[buf], s_sh.at[dstl.at[buf]],
                                  ssc).wait()

        def sums_edge(cur, e):
            ev = jnp.full((LANE,), e, jnp.int32)
            acc1 = jnp.zeros((LANE,), jnp.float32)
            acc2 = jnp.zeros((LANE,), jnp.float32)
            for f in range(NF):
                sf = (bufa[cur, e, pl.ds(f * LANE, LANE)]
                      + bufb[cur, e, pl.ds(f * LANE, LANE)])
                bufa[cur, e, pl.ds(f * LANE, LANE)] = sf
                acc1 = acc1 + sf
                acc2 = acc2 + sf * sf
            c1 = plsc.cumsum(acc1)
            c2 = plsc.cumsum(acc2)
            plsc.store_scatter(st1, [ev], c1, mask=m15)
            plsc.store_scatter(st2, [ev], c2, mask=m15)

        def stats_group(g):
            mu = st1[pl.ds(g, LANE)] * (1.0 / H)
            ms = st2[pl.ds(g, LANE)] * (1.0 / H)
            var = ms - mu * mu + LN_EPS
            yi = magic - lax.shift_right_logical(
                plsc.bitcast(var, jnp.int32), 1)
            y = plsc.bitcast(yi, jnp.float32)
            xh = var * 0.5
            for _ in range(3):
                y = y * (1.5 - xh * y * y)
            st1[pl.ds(g, LANE)] = mu
            st2[pl.ds(g, LANE)] = y

        def norm_edge(cur, e):
            ev = jnp.full((LANE,), e, jnp.int32)
            muv = plsc.load_gather(st1, [ev])
            yv = plsc.load_gather(st2, [ev])
            for f in range(NF):
                sf = bufa[cur, e, pl.ds(f * LANE, LANE)]
                z = (sf - muv) * yv * gvec[f] + bvec[f]
                bufa[cur, e, pl.ds(f * LANE, LANE)] = jnp.maximum(z, 0.0)

        gath(0)

        @pl.loop(0, cpw, step=2)
        def _(j):
            for t in range(2):
                jj = j + t
                cur = t
                nxt = 1 - t

                @pl.when(jj >= 1)
                def _():
                    wait_scat(nxt)

                @pl.when(jj + 1 < cpw)
                def _():
                    wait_idx(nxt)
                    gath(nxt)

                wait_gath(cur)

                for f in range(K // LANE):
                    lidx = dsti[cur, pl.ds(f * LANE, LANE)] - basev
                    dstl[cur, pl.ds(f * LANE, LANE)] = lidx
                    plsc.addupdate_scatter(cntv, [lidx], ones16)

                @plsc.parallel_loop(0, K, unroll=4)
                def _(e):
                    sums_edge(cur, e)

                @plsc.parallel_loop(0, K, step=LANE, unroll=2)
                def _(g):
                    stats_group(g)

                @plsc.parallel_loop(0, K, unroll=4)
                def _(e):
                    norm_edge(cur, e)

                pltpu.async_copy(bufa.at[cur], s_sh.at[dstl.at[cur]],
                                 ssc, add=True)

                @pl.when(jj + 2 < cpw)
                def _():
                    pltpu.async_copy(dsth.at[cid, sid, jj + 2],
                                     dsti.at[cur], sm)
                    pltpu.async_copy(srch.at[cid, sid, jj + 2],
                                     srci.at[cur], sm)

        wait_scat(1)
        pltpu.sync_copy(cntv, outc.at[cid, sid])
        plsc.subcore_barrier()
        pltpu.sync_copy(s_sh.at[pl.ds(sid * RPT, RPT)],
                        out.at[cid, pl.ds(sid * RPT, RPT)])

    return kern


_SC_EDGE = {cpw: _make_sc_edge(cpw) for cpw in (82, 42)}


def _cnt_body(t_ref, o_ref):
    c0 = jnp.sum(t_ref[0], axis=0)
    c1 = jnp.sum(t_ref[1], axis=0)
    o_ref[0:SPLIT, :] = c0[0:SPLIT][:, None]
    o_ref[SPLIT:N, :] = c1[0:N - SPLIT][:, None]


def _tc_cnt(t):
    return pl.pallas_call(
        _cnt_body,
        out_shape=jax.ShapeDtypeStruct((N, 1), jnp.float32),
    )(t)


def _edge_blocks(edges, cpw):
    e = edges.shape[1]
    capc = NSUB * cpw * K
    src = edges[0].astype(jnp.int32)
    dst = edges[1].astype(jnp.int32)
    side = (dst >= SPLIT).astype(jnp.int32)
    pos0 = jnp.cumsum(1 - side) - 1
    pos1 = jnp.cumsum(side) - 1 + capc
    pos = jnp.where(side == 1, pos1, pos0)
    dfill = jnp.concatenate([jnp.full((capc,), SPLIT, jnp.int32),
                             jnp.full((capc,), SPLIT + SPLIT, jnp.int32)])
    dstb = dfill.at[pos].set(dst).reshape(NCORES, NSUB, cpw, K)
    sfill = jnp.full((2 * capc,), N, jnp.int32)
    srcb = sfill.at[pos].set(src).reshape(NCORES, NSUB, cpw, K)
    return srcb, dstb


def _row(v):
    return v.reshape(1, -1)


def _gw_layer(p, h, srcb, dstb, cnt, zz, cpw):
    msg, upd = p["msg"], p["upd"]
    W1 = msg["l1"]["W"]
    A, B, Sself = _tc_pre(h, W1[:H], W1[H:], _row(msg["l1"]["b"]),
                          _row(msg["ln"]["g"]), _row(msg["ln"]["b"]))
    Ap = jnp.pad(A, ((0, NPAD - N), (0, 0)))
    Bp = jnp.pad(B, ((0, NPAD - N), (0, 0)))
    gb = jnp.stack([msg["ln"]["g"], msg["ln"]["b"]])
    S2, C2 = _SC_EDGE[cpw](Ap, Bp, dstb, srcb, zz, gb)
    S = jnp.concatenate([S2[0, :SPLIT], S2[1, :SPLIT]], axis=0)
    if cnt is None:
        cnt = _tc_cnt(C2)
    U1 = upd["l1"]["W"]
    hnew = _tc_post(S, cnt, Sself, h,
                    msg["l2"]["W"], _row(msg["l2"]["b"]),
                    U1[:H], U1[H:], _row(upd["l1"]["b"]),
                    _row(upd["ln"]["g"]), _row(upd["ln"]["b"]),
                    upd["l2"]["W"], _row(upd["l2"]["b"]))
    return hnew, cnt


def kernel(x, fine_edges, coarse_edges, params):
    srcf, dstf = _edge_blocks(fine_edges, 82)
    srcc, dstc = _edge_blocks(coarse_edges, 42)
    zz = jnp.zeros((TPC, SW), jnp.float32)

    enc = params["enc"]
    h = _tc_enc(x, enc["l1"]["W"], _row(enc["l1"]["b"]),
                _row(enc["ln"]["g"]), _row(enc["ln"]["b"]),
                enc["l2"]["W"], _row(enc["l2"]["b"]))

    hf, cf = h, None
    for lp in params["fine"]:
        hf, cf = _gw_layer(lp, hf, srcf, dstf, cf, zz, 82)
    hc, cc = h, None
    for lp in params["coarse"]:
        hc, cc = _gw_layer(lp, hc, srcc, dstc, cc, zz, 42)

    Wm = params["mesh"]["W"]
    return _tc_dec(hf, hc, Wm[:H], Wm[H:], _row(params["mesh"]["b"]),
                   params["dec_l1"]["W"], _row(params["dec_l1"]["b"]),
                   params["dec_l2"]["W"], _row(params["dec_l2"]["b"]))


# speedup vs baseline: 5.5443x; 2.9074x over previous
import dataclasses
import functools

import jax
import jax.numpy as jnp
from jax import lax
from jax.experimental import pallas as pl
from jax.experimental.pallas import tpu as pltpu
from jax.experimental.pallas import tpu_sc as plsc

N = 10000
C = 128
H = 128
FH = 6
LN_EPS = 1e-5

BLK = 2000
NPAD = 10112
SW = 128
K = 64
NCORES = 2
NSUB = 16
NW = NCORES * NSUB
LANE = 16
NF = H // LANE
RPT = NPAD // NSUB


def _ln(t, g, b):
    mu = jnp.mean(t, axis=-1, keepdims=True)
    var = jnp.mean((t - mu) ** 2, axis=-1, keepdims=True)
    return (t - mu) * lax.rsqrt(var + LN_EPS) * g + b


def _rows(i):
    return (i, 0)


def _const(i):
    return (0, 0)


def _mm(a, b):
    return jnp.dot(a, b, preferred_element_type=jnp.float32)


def _enc_body(x_ref, w1_ref, b1_ref, g_ref, bb_ref, w2_ref, c2_ref, o_ref):
    t = _mm(x_ref[:], w1_ref[:]) + b1_ref[:]
    y = jnp.maximum(_ln(t, g_ref[:], bb_ref[:]), 0.0)
    o_ref[:] = _mm(y, w2_ref[:]) + c2_ref[:]


def _tc_enc(x, w1, b1, g, bb, w2, c2):
    return pl.pallas_call(
        _enc_body,
        grid=(N // BLK,),
        in_specs=[
            pl.BlockSpec((BLK, C), _rows),
            pl.BlockSpec((C, H), _const),
            pl.BlockSpec((1, H), _const),
            pl.BlockSpec((1, H), _const),
            pl.BlockSpec((1, H), _const),
            pl.BlockSpec((H, H), _const),
            pl.BlockSpec((1, H), _const),
        ],
        out_specs=pl.BlockSpec((BLK, H), _rows),
        out_shape=jax.ShapeDtypeStruct((N, H), jnp.float32),
    )(x, w1, b1, g, bb, w2, c2)


def _pre_body(h_ref, wa_ref, wb_ref, b1_ref, g_ref, bb_ref,
              a_ref, b_ref, s_ref):
    h = h_ref[:]
    A = _mm(h, wa_ref[:]) + b1_ref[:]
    B = _mm(h, wb_ref[:])
    a_ref[:] = A
    b_ref[:] = B
    s_ref[:] = jnp.maximum(_ln(A + B, g_ref[:], bb_ref[:]), 0.0)


def _tc_pre(h, wa, wb, b1, g, bb):
    return pl.pallas_call(
        _pre_body,
        grid=(N // BLK,),
        in_specs=[
            pl.BlockSpec((BLK, H), _rows),
            pl.BlockSpec((H, H), _const),
            pl.BlockSpec((H, H), _const),
            pl.BlockSpec((1, H), _const),
            pl.BlockSpec((1, H), _const),
            pl.BlockSpec((1, H), _const),
        ],
        out_specs=[
            pl.BlockSpec((BLK, H), _rows),
            pl.BlockSpec((BLK, H), _rows),
            pl.BlockSpec((BLK, H), _rows),
        ],
        out_shape=[
            jax.ShapeDtypeStruct((N, H), jnp.float32),
            jax.ShapeDtypeStruct((N, H), jnp.float32),
            jax.ShapeDtypeStruct((N, H), jnp.float32),
        ],
    )(h, wa, wb, b1, g, bb)


def _post_body(s0_ref, s1_ref, c_ref, ss_ref, h_ref, w2_ref,
               b2_ref, ua_ref, ub_ref, c1_ref, gu_ref, bu_ref, u2_ref,
               c2_ref, o_ref):
    S = s0_ref[:] + s1_ref[:] + ss_ref[:]
    cnt = c_ref[:] + 1.0
    aggr = _mm(S, w2_ref[:]) + cnt * b2_ref[:]
    t = _mm(h_ref[:], ua_ref[:]) + _mm(aggr, ub_ref[:]) + c1_ref[:]
    y = jnp.maximum(_ln(t, gu_ref[:], bu_ref[:]), 0.0)
    o_ref[:] = _mm(y, u2_ref[:]) + c2_ref[:]


def _tc_post(s0, s1, c, ss, h, w2, b2, ua, ub, c1, gu, bu, u2, c2):
    return pl.pallas_call(
        _post_body,
        grid=(N // BLK,),
        in_specs=[
            pl.BlockSpec((BLK, H), _rows),
            pl.BlockSpec((BLK, H), _rows),
            pl.BlockSpec((BLK, 1), _rows),
            pl.BlockSpec((BLK, H), _rows),
            pl.BlockSpec((BLK, H), _rows),
            pl.BlockSpec((H, H), _const),
            pl.BlockSpec((1, H), _const),
            pl.BlockSpec((H, H), _const),
            pl.BlockSpec((H, H), _const),
            pl.BlockSpec((1, H), _const),
            pl.BlockSpec((1, H), _const),
            pl.BlockSpec((1, H), _const),
            pl.BlockSpec((H, H), _const),
            pl.BlockSpec((1, H), _const),
        ],
        out_specs=pl.BlockSpec((BLK, H), _rows),
        out_shape=jax.ShapeDtypeStruct((N, H), jnp.float32),
    )(s0, s1, c, ss, h, w2, b2, ua, ub, c1, gu, bu, u2, c2)


def _dec_body(hf_ref, hc_ref, wma_ref, wmb_ref, bm_ref,
              wd1_ref, bd1_ref, wd2_ref, bd2_ref, o_ref):
    comb = _mm(hf_ref[:], wma_ref[:]) + _mm(hc_ref[:], wmb_ref[:]) + bm_ref[:]
    d = jnp.maximum(_mm(comb, wd1_ref[:]) + bd1_ref[:], 0.0)
    o_ref[:] = _mm(d, wd2_ref[:]) + bd2_ref[:]


def _tc_dec(hf, hc, wma, wmb, bm, wd1, bd1, wd2, bd2):
    OD = FH * C
    return pl.pallas_call(
        _dec_body,
        grid=(N // BLK,),
        in_specs=[
            pl.BlockSpec((BLK, H), _rows),
            pl.BlockSpec((BLK, H), _rows),
            pl.BlockSpec((H, H), _const),
            pl.BlockSpec((H, H), _const),
            pl.BlockSpec((1, H), _const),
            pl.BlockSpec((H, H), _const),
            pl.BlockSpec((1, H), _const),
            pl.BlockSpec((H, OD), _const),
            pl.BlockSpec((1, OD), _const),
        ],
        out_specs=pl.BlockSpec((BLK, OD), _rows),
        out_shape=jax.ShapeDtypeStruct((N, OD), jnp.float32),
    )(hf, hc, wma, wmb, bm, wd1, bd1, wd2, bd2)


def _sc_cp():
    cp = pltpu.CompilerParams()
    if "needs_layout_passes" in pltpu.CompilerParams.__dataclass_fields__:
        cp = dataclasses.replace(cp, needs_layout_passes=False)
    return cp


def _make_sc_edge(cpw):
    mesh = plsc.VectorSubcoreMesh(core_axis_name="c", subcore_axis_name="s")

    @functools.partial(
        pl.kernel,
        out_type=[
            jax.ShapeDtypeStruct((NCORES, NPAD, SW), jnp.float32),
            jax.ShapeDtypeStruct((NW, NPAD), jnp.float32),
        ],
        mesh=mesh,
        compiler_params=_sc_cp(),
        scratch_types=[
            pltpu.VMEM_SHARED((NPAD, SW), jnp.float32),
            pltpu.VMEM((2, K, H), jnp.float32),
            pltpu.VMEM((2, K, H), jnp.float32),
            pltpu.VMEM((2, K), jnp.int32),
            pltpu.VMEM((2, K), jnp.int32),
            pltpu.VMEM((2, K), jnp.int32),
            pltpu.VMEM((2, H), jnp.float32),
            pltpu.VMEM((K,), jnp.float32),
            pltpu.VMEM((K,), jnp.float32),
            pltpu.VMEM((NPAD,), jnp.float32),
            pltpu.SemaphoreType.DMA,
            pltpu.SemaphoreType.DMA,
            pltpu.SemaphoreType.DMA,
            pltpu.SemaphoreType.DMA,
        ],
    )
    def kern(ap, bp, dsth, srch, zz, gb, out, outc, s_sh, bufa, bufb,
             dsti, srci, dstl, gbv, st1, st2, cntv, sga, sgb, sm, ssc):
        cid = lax.axis_index("c")
        sid = lax.axis_index("s")
        w = cid * NSUB + sid

        pltpu.async_copy(gb, gbv, sm).wait()
        pltpu.async_copy(dsth.at[w, 0], dsti.at[0], sm).wait()
        pltpu.async_copy(srch.at[w, 0], srci.at[0], sm).wait()
        pltpu.async_copy(dsth.at[w, 1], dsti.at[1], sm)
        pltpu.async_copy(srch.at[w, 1], srci.at[1], sm)
        pltpu.async_copy(zz.at[pl.ds(sid * RPT, RPT)],
                         s_sh.at[pl.ds(sid * RPT, RPT)], sm).wait()
        plsc.subcore_barrier()

        gvec = [gbv[0, pl.ds(f * LANE, LANE)] for f in range(NF)]
        bvec = [gbv[1, pl.ds(f * LANE, LANE)] for f in range(NF)]
        magic = jnp.full((LANE,), 0x5F3759DF, jnp.int32)
        m15 = lax.iota(jnp.int32, LANE) == (LANE - 1)
        ones16 = jnp.full((LANE,), 1.0, jnp.float32)
        zeros16 = jnp.zeros((LANE,), jnp.float32)

        @plsc.parallel_loop(0, NPAD, step=LANE)
        def _(i):
            cntv[pl.ds(i, LANE)] = zeros16

        def gath(buf):
            pltpu.async_copy(ap.at[dsti.at[buf]], bufa.at[buf], sga)
            pltpu.async_copy(bp.at[srci.at[buf]], bufb.at[buf], sgb)

        def wait_idx(buf):
            pltpu.make_async_copy(dsth.at[w, 0], dsti.at[buf], sm).wait()
            pltpu.make_async_copy(srch.at[w, 0], srci.at[buf], sm).wait()

        def wait_gath(buf):
            pltpu.make_async_copy(ap.at[dsti.at[buf]],
                                  bufa.at[buf], sga).wait()
            pltpu.make_async_copy(bp.at[srci.at[buf]],
                                  bufb.at[buf], sgb).wait()

        def wait_scat(buf):
            pltpu.make_async_copy(bufa.at[buf], s_sh.at[dstl.at[buf]],
                                  ssc).wait()

        def sums_edge(cur, e):
            ev = jnp.full((LANE,), e, jnp.int32)
            acc1 = jnp.zeros((LANE,), jnp.float32)
            acc2 = jnp.zeros((LANE,), jnp.float32)
            for f in range(NF):
                sf = (bufa[cur, e, pl.ds(f * LANE, LANE)]
                      + bufb[cur, e, pl.ds(f * LANE, LANE)])
                bufa[cur, e, pl.ds(f * LANE, LANE)] = sf
                acc1 = acc1 + sf
                acc2 = acc2 + sf * sf
            c1 = plsc.cumsum(acc1)
            c2 = plsc.cumsum(acc2)
            plsc.store_scatter(st1, [ev], c1, mask=m15)
            plsc.store_scatter(st2, [ev], c2, mask=m15)

        def stats_group(g):
            mu = st1[pl.ds(g, LANE)] * (1.0 / H)
            ms = st2[pl.ds(g, LANE)] * (1.0 / H)
            var = ms - mu * mu + LN_EPS
            yi = magic - lax.shift_right_logical(
                plsc.bitcast(var, jnp.int32), 1)
            y = plsc.bitcast(yi, jnp.float32)
            xh = var * 0.5
            for _ in range(3):
                y = y * (1.5 - xh * y * y)
            st1[pl.ds(g, LANE)] = mu
            st2[pl.ds(g, LANE)] = y

        def norm_edge(cur, e):
            ev = jnp.full((LANE,), e, jnp.int32)
            muv = plsc.load_gather(st1, [ev])
            yv = plsc.load_gather(st2, [ev])
            for f in range(NF):
                sf = bufa[cur, e, pl.ds(f * LANE, LANE)]
                z = (sf - muv) * yv * gvec[f] + bvec[f]
                bufa[cur, e, pl.ds(f * LANE, LANE)] = jnp.maximum(z, 0.0)

        gath(0)

        @pl.loop(0, cpw, step=2)
        def _(j):
            for t in range(2):
                jj = j + t
                cur = t
                nxt = 1 - t

                @pl.when(jj >= 1)
                def _():
                    wait_scat(nxt)

                @pl.when(jj + 1 < cpw)
                def _():
                    wait_idx(nxt)
                    gath(nxt)

                wait_gath(cur)

                for f in range(K // LANE):
                    lidx = dsti[cur, pl.ds(f * LANE, LANE)]
                    dstl[cur, pl.ds(f * LANE, LANE)] = lidx
                    plsc.addupdate_scatter(cntv, [lidx], ones16)

                @plsc.parallel_loop(0, K, unroll=4)
                def _(e):
                    sums_edge(cur, e)

                @plsc.parallel_loop(0, K, step=LANE, unroll=2)
                def _(g):
                    stats_group(g)

                @plsc.parallel_loop(0, K, unroll=4)
                def _(e):
                    norm_edge(cur, e)

                pltpu.async_copy(bufa.at[cur], s_sh.at[dstl.at[cur]],
                                 ssc, add=True)

                @pl.when(jj + 2 < cpw)
                def _():
                    pltpu.async_copy(dsth.at[w, jj + 2], dsti.at[cur], sm)
                    pltpu.async_copy(srch.at[w, jj + 2], srci.at[cur], sm)

        wait_scat(1)
        pltpu.sync_copy(cntv, outc.at[w])
        plsc.subcore_barrier()
        pltpu.sync_copy(s_sh.at[pl.ds(sid * RPT, RPT)],
                        out.at[cid, pl.ds(sid * RPT, RPT)])

    return kern


CPW_FINE = 158
CPW_COARSE = 80
_SC_EDGE = {cpw: _make_sc_edge(cpw) for cpw in (CPW_FINE, CPW_COARSE)}


def _cnt_body(t_ref, o_ref):
    c = jnp.sum(t_ref[:], axis=0)
    o_ref[:] = c[0:N][:, None]


def _tc_cnt(t):
    return pl.pallas_call(
        _cnt_body,
        out_shape=jax.ShapeDtypeStruct((N, 1), jnp.float32),
    )(t)


def _edge_blocks(edges, cpw):
    e = edges.shape[1]
    cap = NW * cpw * K
    src = edges[0].astype(jnp.int32)
    dst = edges[1].astype(jnp.int32)
    pad = jnp.full((cap - e,), N, jnp.int32)
    srcb = jnp.concatenate([src, pad]).reshape(NW, cpw, K)
    dstb = jnp.concatenate([dst, pad]).reshape(NW, cpw, K)
    return srcb, dstb


def _row(v):
    return v.reshape(1, -1)


def _gw_layer(p, h, srcb, dstb, cnt, zz, cpw):
    msg, upd = p["msg"], p["upd"]
    W1 = msg["l1"]["W"]
    A, B, Sself = _tc_pre(h, W1[:H], W1[H:], _row(msg["l1"]["b"]),
                          _row(msg["ln"]["g"]), _row(msg["ln"]["b"]))
    Ap = jnp.pad(A, ((0, NPAD - N), (0, 0)))
    Bp = jnp.pad(B, ((0, NPAD - N), (0, 0)))
    gb = jnp.stack([msg["ln"]["g"], msg["ln"]["b"]])
    S2, C2 = _SC_EDGE[cpw](Ap, Bp, dstb, srcb, zz, gb)
    if cnt is None:
        cnt = _tc_cnt(C2)
    U1 = upd["l1"]["W"]
    hnew = _tc_post(S2[0, :N], S2[1, :N], cnt, Sself, h,
                    msg["l2"]["W"], _row(msg["l2"]["b"]),
                    U1[:H], U1[H:], _row(upd["l1"]["b"]),
                    _row(upd["ln"]["g"]), _row(upd["ln"]["b"]),
                    upd["l2"]["W"], _row(upd["l2"]["b"]))
    return hnew, cnt


def kernel(x, fine_edges, coarse_edges, params):
    srcf, dstf = _edge_blocks(fine_edges, CPW_FINE)
    srcc, dstc = _edge_blocks(coarse_edges, CPW_COARSE)
    zz = jnp.zeros((NPAD, SW), jnp.float32)

    enc = params["enc"]
    h = _tc_enc(x, enc["l1"]["W"], _row(enc["l1"]["b"]),
                _row(enc["ln"]["g"]), _row(enc["ln"]["b"]),
                enc["l2"]["W"], _row(enc["l2"]["b"]))

    hf, cf = h, None
    for lp in params["fine"]:
        hf, cf = _gw_layer(lp, hf, srcf, dstf, cf, zz, CPW_FINE)
    hc, cc = h, None
    for lp in params["coarse"]:
        hc, cc = _gw_layer(lp, hc, srcc, dstc, cc, zz, CPW_COARSE)

    Wm = params["mesh"]["W"]
    return _tc_dec(hf, hc, Wm[:H], Wm[H:], _row(params["mesh"]["b"]),
                   params["dec_l1"]["W"], _row(params["dec_l1"]["b"]),
                   params["dec_l2"]["W"], _row(params["dec_l2"]["b"]))


# speedup vs baseline: 5.7921x; 1.0447x over previous
import dataclasses
import functools

import jax
import jax.numpy as jnp
from jax import lax
from jax.experimental import pallas as pl
from jax.experimental.pallas import tpu as pltpu
from jax.experimental.pallas import tpu_sc as plsc

N = 10000
C = 128
H = 128
FH = 6
LN_EPS = 1e-5

BLK = 2000
BLKP = 1264
NPAD = 10112
SW = 128
K = 64
NCORES = 2
NSUB = 16
NW = NCORES * NSUB
LANE = 16
NF = H // LANE
RPT = NPAD // NSUB


def _ln(t, g, b):
    mu = jnp.mean(t, axis=-1, keepdims=True)
    var = jnp.mean((t - mu) ** 2, axis=-1, keepdims=True)
    return (t - mu) * lax.rsqrt(var + LN_EPS) * g + b


def _rows(i):
    return (i, 0)


def _const(i):
    return (0, 0)


def _mm(a, b):
    return jnp.dot(a, b, preferred_element_type=jnp.float32)


def _enc_body(x_ref, w1_ref, b1_ref, g_ref, bb_ref, w2_ref, c2_ref, o_ref):
    t = _mm(x_ref[:], w1_ref[:]) + b1_ref[:]
    y = jnp.maximum(_ln(t, g_ref[:], bb_ref[:]), 0.0)
    o_ref[:] = _mm(y, w2_ref[:]) + c2_ref[:]


def _tc_enc(x, w1, b1, g, bb, w2, c2):
    return pl.pallas_call(
        _enc_body,
        grid=(NPAD // BLKP,),
        in_specs=[
            pl.BlockSpec((BLKP, C), _rows),
            pl.BlockSpec((C, H), _const),
            pl.BlockSpec((1, H), _const),
            pl.BlockSpec((1, H), _const),
            pl.BlockSpec((1, H), _const),
            pl.BlockSpec((H, H), _const),
            pl.BlockSpec((1, H), _const),
        ],
        out_specs=pl.BlockSpec((BLKP, H), _rows),
        out_shape=jax.ShapeDtypeStruct((NPAD, H), jnp.float32),
    )(x, w1, b1, g, bb, w2, c2)


def _pre_body(h_ref, wa_ref, wb_ref, b1_ref, g_ref, bb_ref,
              a_ref, b_ref, s_ref):
    h = h_ref[:]
    A = _mm(h, wa_ref[:]) + b1_ref[:]
    B = _mm(h, wb_ref[:])
    a_ref[:] = A
    b_ref[:] = B
    s_ref[:] = jnp.maximum(_ln(A + B, g_ref[:], bb_ref[:]), 0.0)


def _tc_pre(h, wa, wb, b1, g, bb):
    return pl.pallas_call(
        _pre_body,
        grid=(NPAD // BLKP,),
        in_specs=[
            pl.BlockSpec((BLKP, H), _rows),
            pl.BlockSpec((H, H), _const),
            pl.BlockSpec((H, H), _const),
            pl.BlockSpec((1, H), _const),
            pl.BlockSpec((1, H), _const),
            pl.BlockSpec((1, H), _const),
        ],
        out_specs=[
            pl.BlockSpec((BLKP, H), _rows),
            pl.BlockSpec((BLKP, H), _rows),
            pl.BlockSpec((BLKP, H), _rows),
        ],
        out_shape=[
            jax.ShapeDtypeStruct((NPAD, H), jnp.float32),
            jax.ShapeDtypeStruct((NPAD, H), jnp.float32),
            jax.ShapeDtypeStruct((NPAD, H), jnp.float32),
        ],
    )(h, wa, wb, b1, g, bb)


def _post_body(s0_ref, s1_ref, c_ref, ss_ref, h_ref, w2_ref,
               b2_ref, ua_ref, ub_ref, c1_ref, gu_ref, bu_ref, u2_ref,
               c2_ref, o_ref):
    S = s0_ref[:] + s1_ref[:] + ss_ref[:]
    cnt = c_ref[:] + 1.0
    aggr = _mm(S, w2_ref[:]) + cnt * b2_ref[:]
    t = _mm(h_ref[:], ua_ref[:]) + _mm(aggr, ub_ref[:]) + c1_ref[:]
    y = jnp.maximum(_ln(t, gu_ref[:], bu_ref[:]), 0.0)
    o_ref[:] = _mm(y, u2_ref[:]) + c2_ref[:]


def _tc_post(s0, s1, c, ss, h, w2, b2, ua, ub, c1, gu, bu, u2, c2):
    return pl.pallas_call(
        _post_body,
        grid=(NPAD // BLKP,),
        in_specs=[
            pl.BlockSpec((BLKP, H), _rows),
            pl.BlockSpec((BLKP, H), _rows),
            pl.BlockSpec((BLKP, 1), _rows),
            pl.BlockSpec((BLKP, H), _rows),
            pl.BlockSpec((BLKP, H), _rows),
            pl.BlockSpec((H, H), _const),
            pl.BlockSpec((1, H), _const),
            pl.BlockSpec((H, H), _const),
            pl.BlockSpec((H, H), _const),
            pl.BlockSpec((1, H), _const),
            pl.BlockSpec((1, H), _const),
            pl.BlockSpec((1, H), _const),
            pl.BlockSpec((H, H), _const),
            pl.BlockSpec((1, H), _const),
        ],
        out_specs=pl.BlockSpec((BLKP, H), _rows),
        out_shape=jax.ShapeDtypeStruct((NPAD, H), jnp.float32),
    )(s0, s1, c, ss, h, w2, b2, ua, ub, c1, gu, bu, u2, c2)


def _dec_body(hf_ref, hc_ref, wma_ref, wmb_ref, bm_ref,
              wd1_ref, bd1_ref, wd2_ref, bd2_ref, o_ref):
    comb = _mm(hf_ref[:], wma_ref[:]) + _mm(hc_ref[:], wmb_ref[:]) + bm_ref[:]
    d = jnp.maximum(_mm(comb, wd1_ref[:]) + bd1_ref[:], 0.0)
    o_ref[:] = _mm(d, wd2_ref[:]) + bd2_ref[:]


def _tc_dec(hf, hc, wma, wmb, bm, wd1, bd1, wd2, bd2):
    OD = FH * C
    return pl.pallas_call(
        _dec_body,
        grid=(N // BLK,),
        in_specs=[
            pl.BlockSpec((BLK, H), _rows),
            pl.BlockSpec((BLK, H), _rows),
            pl.BlockSpec((H, H), _const),
            pl.BlockSpec((H, H), _const),
            pl.BlockSpec((1, H), _const),
            pl.BlockSpec((H, H), _const),
            pl.BlockSpec((1, H), _const),
            pl.BlockSpec((H, OD), _const),
            pl.BlockSpec((1, OD), _const),
        ],
        out_specs=pl.BlockSpec((BLK, OD), _rows),
        out_shape=jax.ShapeDtypeStruct((N, OD), jnp.float32),
    )(hf, hc, wma, wmb, bm, wd1, bd1, wd2, bd2)


def _sc_cp():
    cp = pltpu.CompilerParams()
    if "needs_layout_passes" in pltpu.CompilerParams.__dataclass_fields__:
        cp = dataclasses.replace(cp, needs_layout_passes=False)
    return cp


def _make_sc_edge(cpw):
    mesh = plsc.VectorSubcoreMesh(core_axis_name="c", subcore_axis_name="s")

    @functools.partial(
        pl.kernel,
        out_type=[
            jax.ShapeDtypeStruct((NCORES, NPAD, SW), jnp.float32),
            jax.ShapeDtypeStruct((NW, NPAD), jnp.float32),
        ],
        mesh=mesh,
        compiler_params=_sc_cp(),
        scratch_types=[
            pltpu.VMEM_SHARED((NPAD, SW), jnp.float32),
            pltpu.VMEM((2, K, H), jnp.float32),
            pltpu.VMEM((2, K, H), jnp.float32),
            pltpu.VMEM((2, K), jnp.int32),
            pltpu.VMEM((2, K), jnp.int32),
            pltpu.VMEM((2, K), jnp.int32),
            pltpu.VMEM((2, H), jnp.float32),
            pltpu.VMEM((K,), jnp.float32),
            pltpu.VMEM((K,), jnp.float32),
            pltpu.VMEM((NPAD,), jnp.float32),
            pltpu.SemaphoreType.DMA,
            pltpu.SemaphoreType.DMA,
            pltpu.SemaphoreType.DMA,
            pltpu.SemaphoreType.DMA,
        ],
    )
    def kern(ap, bp, dsth, srch, zz, gb, out, outc, s_sh, bufa, bufb,
             dsti, srci, dstl, gbv, st1, st2, cntv, sga, sgb, sm, ssc):
        cid = lax.axis_index("c")
        sid = lax.axis_index("s")
        w = cid * NSUB + sid

        pltpu.async_copy(gb, gbv, sm).wait()
        pltpu.async_copy(dsth.at[w, 0], dsti.at[0], sm).wait()
        pltpu.async_copy(srch.at[w, 0], srci.at[0], sm).wait()
        pltpu.async_copy(dsth.at[w, 1], dsti.at[1], sm)
        pltpu.async_copy(srch.at[w, 1], srci.at[1], sm)
        pltpu.async_copy(zz.at[pl.ds(sid * RPT, RPT)],
                         s_sh.at[pl.ds(sid * RPT, RPT)], sm).wait()
        plsc.subcore_barrier()

        gvec = [gbv[0, pl.ds(f * LANE, LANE)] for f in range(NF)]
        bvec = [gbv[1, pl.ds(f * LANE, LANE)] for f in range(NF)]
        magic = jnp.full((LANE,), 0x5F3759DF, jnp.int32)
        m15 = lax.iota(jnp.int32, LANE) == (LANE - 1)
        ones16 = jnp.full((LANE,), 1.0, jnp.float32)
        zeros16 = jnp.zeros((LANE,), jnp.float32)

        @plsc.parallel_loop(0, NPAD, step=LANE)
        def _(i):
            cntv[pl.ds(i, LANE)] = zeros16

        def gath(buf):
            pltpu.async_copy(ap.at[dsti.at[buf]], bufa.at[buf], sga)
            pltpu.async_copy(bp.at[srci.at[buf]], bufb.at[buf], sgb)

        def wait_idx(buf):
            pltpu.make_async_copy(dsth.at[w, 0], dsti.at[buf], sm).wait()
            pltpu.make_async_copy(srch.at[w, 0], srci.at[buf], sm).wait()

        def wait_gath(buf):
            pltpu.make_async_copy(ap.at[dsti.at[buf]],
                                  bufa.at[buf], sga).wait()
            pltpu.make_async_copy(bp.at[srci.at[buf]],
                                  bufb.at[buf], sgb).wait()

        def wait_scat(buf):
            pltpu.make_async_copy(bufa.at[buf], s_sh.at[dstl.at[buf]],
                                  ssc).wait()

        def sums_edge(cur, e):
            ev = jnp.full((LANE,), e, jnp.int32)
            acc1 = jnp.zeros((LANE,), jnp.float32)
            acc2 = jnp.zeros((LANE,), jnp.float32)
            for f in range(NF):
                sf = (bufa[cur, e, pl.ds(f * LANE, LANE)]
                      + bufb[cur, e, pl.ds(f * LANE, LANE)])
                bufa[cur, e, pl.ds(f * LANE, LANE)] = sf
                acc1 = acc1 + sf
                acc2 = acc2 + sf * sf
            c1 = plsc.cumsum(acc1)
            c2 = plsc.cumsum(acc2)
            plsc.store_scatter(st1, [ev], c1, mask=m15)
            plsc.store_scatter(st2, [ev], c2, mask=m15)

        def stats_group(g):
            mu = st1[pl.ds(g, LANE)] * (1.0 / H)
            ms = st2[pl.ds(g, LANE)] * (1.0 / H)
            var = ms - mu * mu + LN_EPS
            yi = magic - lax.shift_right_logical(
                plsc.bitcast(var, jnp.int32), 1)
            y = plsc.bitcast(yi, jnp.float32)
            xh = var * 0.5
            for _ in range(3):
                y = y * (1.5 - xh * y * y)
            st1[pl.ds(g, LANE)] = mu
            st2[pl.ds(g, LANE)] = y

        def norm_edge(cur, e):
            ev = jnp.full((LANE,), e, jnp.int32)
            muv = plsc.load_gather(st1, [ev])
            yv = plsc.load_gather(st2, [ev])
            for f in range(NF):
                sf = bufa[cur, e, pl.ds(f * LANE, LANE)]
                z = (sf - muv) * yv * gvec[f] + bvec[f]
                bufa[cur, e, pl.ds(f * LANE, LANE)] = jnp.maximum(z, 0.0)

        gath(0)

        @pl.loop(0, cpw, step=2)
        def _(j):
            for t in range(2):
                jj = j + t
                cur = t
                nxt = 1 - t

                @pl.when(jj >= 1)
                def _():
                    wait_scat(nxt)

                @pl.when(jj + 1 < cpw)
                def _():
                    wait_idx(nxt)
                    gath(nxt)

                wait_gath(cur)

                for f in range(K // LANE):
                    lidx = dsti[cur, pl.ds(f * LANE, LANE)]
                    dstl[cur, pl.ds(f * LANE, LANE)] = lidx
                    plsc.addupdate_scatter(cntv, [lidx], ones16)

                @plsc.parallel_loop(0, K, unroll=4)
                def _(e):
                    sums_edge(cur, e)

                @plsc.parallel_loop(0, K, step=LANE, unroll=2)
                def _(g):
                    stats_group(g)

                @plsc.parallel_loop(0, K, unroll=4)
                def _(e):
                    norm_edge(cur, e)

                pltpu.async_copy(bufa.at[cur], s_sh.at[dstl.at[cur]],
                                 ssc, add=True)

                @pl.when(jj + 2 < cpw)
                def _():
                    pltpu.async_copy(dsth.at[w, jj + 2], dsti.at[cur], sm)
                    pltpu.async_copy(srch.at[w, jj + 2], srci.at[cur], sm)

        wait_scat(1)
        pltpu.sync_copy(cntv, outc.at[w])
        plsc.subcore_barrier()
        pltpu.sync_copy(s_sh.at[pl.ds(sid * RPT, RPT)],
                        out.at[cid, pl.ds(sid * RPT, RPT)])

    return kern


CPW_FINE = 158
CPW_COARSE = 80
_SC_EDGE = {cpw: _make_sc_edge(cpw) for cpw in (CPW_FINE, CPW_COARSE)}


def _cnt_body(t_ref, o_ref):
    c = jnp.sum(t_ref[:], axis=0)
    o_ref[:] = c[:, None]


def _tc_cnt(t):
    return pl.pallas_call(
        _cnt_body,
        out_shape=jax.ShapeDtypeStruct((NPAD, 1), jnp.float32),
    )(t)


def _edge_blocks(edges, cpw):
    e = edges.shape[1]
    cap = NW * cpw * K
    src = edges[0].astype(jnp.int32)
    dst = edges[1].astype(jnp.int32)
    pad = jnp.full((cap - e,), N, jnp.int32)
    srcb = jnp.concatenate([src, pad]).reshape(NW, cpw, K)
    dstb = jnp.concatenate([dst, pad]).reshape(NW, cpw, K)
    return srcb, dstb


def _row(v):
    return v.reshape(1, -1)


def _gw_layer(p, h, srcb, dstb, cnt, zz, cpw):
    msg, upd = p["msg"], p["upd"]
    W1 = msg["l1"]["W"]
    A, B, Sself = _tc_pre(h, W1[:H], W1[H:], _row(msg["l1"]["b"]),
                          _row(msg["ln"]["g"]), _row(msg["ln"]["b"]))
    gb = jnp.stack([msg["ln"]["g"], msg["ln"]["b"]])
    S2, C2 = _SC_EDGE[cpw](A, B, dstb, srcb, zz, gb)
    if cnt is None:
        cnt = _tc_cnt(C2)
    U1 = upd["l1"]["W"]
    hnew = _tc_post(S2[0], S2[1], cnt, Sself, h,
                    msg["l2"]["W"], _row(msg["l2"]["b"]),
                    U1[:H], U1[H:], _row(upd["l1"]["b"]),
                    _row(upd["ln"]["g"]), _row(upd["ln"]["b"]),
                    upd["l2"]["W"], _row(upd["l2"]["b"]))
    return hnew, cnt


def kernel(x, fine_edges, coarse_edges, params):
    srcf, dstf = _edge_blocks(fine_edges, CPW_FINE)
    srcc, dstc = _edge_blocks(coarse_edges, CPW_COARSE)
    zz = jnp.zeros((NPAD, SW), jnp.float32)

    xp = jnp.pad(x, ((0, NPAD - N), (0, 0)))
    enc = params["enc"]
    h = _tc_enc(xp, enc["l1"]["W"], _row(enc["l1"]["b"]),
                _row(enc["ln"]["g"]), _row(enc["ln"]["b"]),
                enc["l2"]["W"], _row(enc["l2"]["b"]))

    hf, cf = h, None
    for lp in params["fine"]:
        hf, cf = _gw_layer(lp, hf, srcf, dstf, cf, zz, CPW_FINE)
    hc, cc = h, None
    for lp in params["coarse"]:
        hc, cc = _gw_layer(lp, hc, srcc, dstc, cc, zz, CPW_COARSE)

    Wm = params["mesh"]["W"]
    return _tc_dec(hf, hc, Wm[:H], Wm[H:], _row(params["mesh"]["b"]),
                   params["dec_l1"]["W"], _row(params["dec_l1"]["b"]),
                   params["dec_l2"]["W"], _row(params["dec_l2"]["b"]))


# speedup vs baseline: 5.7974x; 1.0009x over previous
import dataclasses
import functools

import jax
import jax.numpy as jnp
from jax import lax
from jax.experimental import pallas as pl
from jax.experimental.pallas import tpu as pltpu
from jax.experimental.pallas import tpu_sc as plsc

N = 10000
C = 128
H = 128
FH = 6
LN_EPS = 1e-5

BLK = 2000
BLKP = 1264
NPAD = 10112
SW = 128
K = 64
NCORES = 2
NSUB = 16
NW = NCORES * NSUB
LANE = 16
NF = H // LANE
RPT = NPAD // NSUB


def _ln(t, g, b):
    mu = jnp.mean(t, axis=-1, keepdims=True)
    var = jnp.mean((t - mu) ** 2, axis=-1, keepdims=True)
    return (t - mu) * lax.rsqrt(var + LN_EPS) * g + b


def _rows(i):
    return (i, 0)


def _const(i):
    return (0, 0)


def _mm(a, b):
    return jnp.dot(a, b, preferred_element_type=jnp.float32)


def _enc_body(x_ref, w1_ref, b1_ref, g_ref, bb_ref, w2_ref, c2_ref, o_ref):
    t = _mm(x_ref[:], w1_ref[:]) + b1_ref[:]
    y = jnp.maximum(_ln(t, g_ref[:], bb_ref[:]), 0.0)
    o_ref[:] = _mm(y, w2_ref[:]) + c2_ref[:]


def _tc_enc(x, w1, b1, g, bb, w2, c2):
    return pl.pallas_call(
        _enc_body,
        grid=(NPAD // BLKP,),
        in_specs=[
            pl.BlockSpec((BLKP, C), _rows),
            pl.BlockSpec((C, H), _const),
            pl.BlockSpec((1, H), _const),
            pl.BlockSpec((1, H), _const),
            pl.BlockSpec((1, H), _const),
            pl.BlockSpec((H, H), _const),
            pl.BlockSpec((1, H), _const),
        ],
        out_specs=pl.BlockSpec((BLKP, H), _rows),
        out_shape=jax.ShapeDtypeStruct((NPAD, H), jnp.float32),
    )(x, w1, b1, g, bb, w2, c2)


def _pre_body(h_ref, wa_ref, wb_ref, b1_ref, g_ref, bb_ref,
              a_ref, b_ref, s_ref):
    h = h_ref[:]
    A = _mm(h, wa_ref[:]) + b1_ref[:]
    B = _mm(h, wb_ref[:])
    a_ref[:] = A
    b_ref[:] = B
    s_ref[:] = jnp.maximum(_ln(A + B, g_ref[:], bb_ref[:]), 0.0)


def _tc_pre(h, wa, wb, b1, g, bb):
    return pl.pallas_call(
        _pre_body,
        grid=(NPAD // BLKP,),
        in_specs=[
            pl.BlockSpec((BLKP, H), _rows),
            pl.BlockSpec((H, H), _const),
            pl.BlockSpec((H, H), _const),
            pl.BlockSpec((1, H), _const),
            pl.BlockSpec((1, H), _const),
            pl.BlockSpec((1, H), _const),
        ],
        out_specs=[
            pl.BlockSpec((BLKP, H), _rows),
            pl.BlockSpec((BLKP, H), _rows),
            pl.BlockSpec((BLKP, H), _rows),
        ],
        out_shape=[
            jax.ShapeDtypeStruct((NPAD, H), jnp.float32),
            jax.ShapeDtypeStruct((NPAD, H), jnp.float32),
            jax.ShapeDtypeStruct((NPAD, H), jnp.float32),
        ],
    )(h, wa, wb, b1, g, bb)


def _post_body(s0_ref, s1_ref, c_ref, ss_ref, h_ref, w2_ref,
               b2_ref, ua_ref, ub_ref, c1_ref, gu_ref, bu_ref, u2_ref,
               c2_ref, o_ref):
    S = s0_ref[:] + s1_ref[:] + ss_ref[:]
    cnt = c_ref[:] + 1.0
    aggr = _mm(S, w2_ref[:]) + cnt * b2_ref[:]
    t = _mm(h_ref[:], ua_ref[:]) + _mm(aggr, ub_ref[:]) + c1_ref[:]
    y = jnp.maximum(_ln(t, gu_ref[:], bu_ref[:]), 0.0)
    o_ref[:] = _mm(y, u2_ref[:]) + c2_ref[:]


def _tc_post(s0, s1, c, ss, h, w2, b2, ua, ub, c1, gu, bu, u2, c2):
    return pl.pallas_call(
        _post_body,
        grid=(NPAD // BLKP,),
        in_specs=[
            pl.BlockSpec((BLKP, H), _rows),
            pl.BlockSpec((BLKP, H), _rows),
            pl.BlockSpec((BLKP, 1), _rows),
            pl.BlockSpec((BLKP, H), _rows),
            pl.BlockSpec((BLKP, H), _rows),
            pl.BlockSpec((H, H), _const),
            pl.BlockSpec((1, H), _const),
            pl.BlockSpec((H, H), _const),
            pl.BlockSpec((H, H), _const),
            pl.BlockSpec((1, H), _const),
            pl.BlockSpec((1, H), _const),
            pl.BlockSpec((1, H), _const),
            pl.BlockSpec((H, H), _const),
            pl.BlockSpec((1, H), _const),
        ],
        out_specs=pl.BlockSpec((BLKP, H), _rows),
        out_shape=jax.ShapeDtypeStruct((NPAD, H), jnp.float32),
    )(s0, s1, c, ss, h, w2, b2, ua, ub, c1, gu, bu, u2, c2)


def _dec_body(hf_ref, hc_ref, wma_ref, wmb_ref, bm_ref,
              wd1_ref, bd1_ref, wd2_ref, bd2_ref, o_ref):
    comb = _mm(hf_ref[:], wma_ref[:]) + _mm(hc_ref[:], wmb_ref[:]) + bm_ref[:]
    d = jnp.maximum(_mm(comb, wd1_ref[:]) + bd1_ref[:], 0.0)
    o_ref[:] = _mm(d, wd2_ref[:]) + bd2_ref[:]


def _tc_dec(hf, hc, wma, wmb, bm, wd1, bd1, wd2, bd2):
    OD = FH * C
    return pl.pallas_call(
        _dec_body,
        grid=(N // BLK,),
        in_specs=[
            pl.BlockSpec((BLK, H), _rows),
            pl.BlockSpec((BLK, H), _rows),
            pl.BlockSpec((H, H), _const),
            pl.BlockSpec((H, H), _const),
            pl.BlockSpec((1, H), _const),
            pl.BlockSpec((H, H), _const),
            pl.BlockSpec((1, H), _const),
            pl.BlockSpec((H, OD), _const),
            pl.BlockSpec((1, OD), _const),
        ],
        out_specs=pl.BlockSpec((BLK, OD), _rows),
        out_shape=jax.ShapeDtypeStruct((N, OD), jnp.float32),
    )(hf, hc, wma, wmb, bm, wd1, bd1, wd2, bd2)


def _sc_cp():
    cp = pltpu.CompilerParams()
    if "needs_layout_passes" in pltpu.CompilerParams.__dataclass_fields__:
        cp = dataclasses.replace(cp, needs_layout_passes=False)
    return cp


def _make_sc_edge(cpw):
    mesh = plsc.VectorSubcoreMesh(core_axis_name="c", subcore_axis_name="s")

    @functools.partial(
        pl.kernel,
        out_type=[
            jax.ShapeDtypeStruct((NCORES, NPAD, SW), jnp.float32),
            jax.ShapeDtypeStruct((NW, NPAD), jnp.float32),
        ],
        mesh=mesh,
        compiler_params=_sc_cp(),
        scratch_types=[
            pltpu.VMEM_SHARED((NPAD, SW), jnp.float32),
            pltpu.VMEM((2, K, H), jnp.float32),
            pltpu.VMEM((2, K, H), jnp.float32),
            pltpu.VMEM((2, K), jnp.int32),
            pltpu.VMEM((2, K), jnp.int32),
            pltpu.VMEM((2, K), jnp.int32),
            pltpu.VMEM((2, H), jnp.float32),
            pltpu.VMEM((K,), jnp.float32),
            pltpu.VMEM((K,), jnp.float32),
            pltpu.VMEM((NPAD,), jnp.float32),
            pltpu.SemaphoreType.DMA,
            pltpu.SemaphoreType.DMA,
            pltpu.SemaphoreType.DMA,
            pltpu.SemaphoreType.DMA,
        ],
    )
    def kern(ap, bp, dsth, srch, zz, gb, out, outc, s_sh, bufa, bufb,
             dsti, srci, dstl, gbv, st1, st2, cntv, sga, sgb, sm, ssc):
        cid = lax.axis_index("c")
        sid = lax.axis_index("s")
        w = cid * NSUB + sid

        pltpu.async_copy(gb, gbv, sm).wait()
        pltpu.async_copy(dsth.at[w, 0], dsti.at[0], sm).wait()
        pltpu.async_copy(srch.at[w, 0], srci.at[0], sm).wait()
        pltpu.async_copy(dsth.at[w, 1], dsti.at[1], sm)
        pltpu.async_copy(srch.at[w, 1], srci.at[1], sm)
        pltpu.async_copy(zz.at[pl.ds(sid * RPT, RPT)],
                         s_sh.at[pl.ds(sid * RPT, RPT)], sm).wait()
        plsc.subcore_barrier()

        gvec = [gbv[0, pl.ds(f * LANE, LANE)] for f in range(NF)]
        bvec = [gbv[1, pl.ds(f * LANE, LANE)] for f in range(NF)]
        magic = jnp.full((LANE,), 0x5F3759DF, jnp.int32)
        m15 = lax.iota(jnp.int32, LANE) == (LANE - 1)
        ones16 = jnp.full((LANE,), 1.0, jnp.float32)
        zeros16 = jnp.zeros((LANE,), jnp.float32)

        @plsc.parallel_loop(0, NPAD, step=LANE)
        def _(i):
            cntv[pl.ds(i, LANE)] = zeros16

        def gath(buf):
            pltpu.async_copy(ap.at[dsti.at[buf]], bufa.at[buf], sga)
            pltpu.async_copy(bp.at[srci.at[buf]], bufb.at[buf], sgb)

        def wait_idx(buf):
            pltpu.make_async_copy(dsth.at[w, 0], dsti.at[buf], sm).wait()
            pltpu.make_async_copy(srch.at[w, 0], srci.at[buf], sm).wait()

        def wait_gath(buf):
            pltpu.make_async_copy(ap.at[dsti.at[buf]],
                                  bufa.at[buf], sga).wait()
            pltpu.make_async_copy(bp.at[srci.at[buf]],
                                  bufb.at[buf], sgb).wait()

        def wait_scat(buf):
            pltpu.make_async_copy(bufa.at[buf], s_sh.at[dstl.at[buf]],
                                  ssc).wait()

        def sums_edge(cur, e):
            ev = jnp.full((LANE,), e, jnp.int32)
            acc1 = jnp.zeros((LANE,), jnp.float32)
            acc2 = jnp.zeros((LANE,), jnp.float32)
            for f in range(NF):
                sf = (bufa[cur, e, pl.ds(f * LANE, LANE)]
                      + bufb[cur, e, pl.ds(f * LANE, LANE)])
                bufa[cur, e, pl.ds(f * LANE, LANE)] = sf
                acc1 = acc1 + sf
                acc2 = acc2 + sf * sf
            c1 = plsc.cumsum(acc1)
            c2 = plsc.cumsum(acc2)
            plsc.store_scatter(st1, [ev], c1, mask=m15)
            plsc.store_scatter(st2, [ev], c2, mask=m15)

        def stats_group(g):
            mu = st1[pl.ds(g, LANE)] * (1.0 / H)
            ms = st2[pl.ds(g, LANE)] * (1.0 / H)
            var = ms - mu * mu + LN_EPS
            yi = magic - lax.shift_right_logical(
                plsc.bitcast(var, jnp.int32), 1)
            y = plsc.bitcast(yi, jnp.float32)
            xh = var * 0.5
            for _ in range(3):
                y = y * (1.5 - xh * y * y)
            st1[pl.ds(g, LANE)] = mu
            st2[pl.ds(g, LANE)] = y

        def norm_edge(cur, e):
            ev = jnp.full((LANE,), e, jnp.int32)
            muv = plsc.load_gather(st1, [ev])
            yv = plsc.load_gather(st2, [ev])
            for f in range(NF):
                sf = bufa[cur, e, pl.ds(f * LANE, LANE)]
                z = (sf - muv) * yv * gvec[f] + bvec[f]
                bufa[cur, e, pl.ds(f * LANE, LANE)] = jnp.maximum(z, 0.0)

        gath(0)

        @pl.loop(0, cpw, step=2)
        def _(j):
            for t in range(2):
                jj = j + t
                cur = t
                nxt = 1 - t

                @pl.when(jj >= 1)
                def _():
                    wait_scat(nxt)

                @pl.when(jj + 1 < cpw)
                def _():
                    wait_idx(nxt)
                    gath(nxt)

                wait_gath(cur)

                for f in range(K // LANE):
                    lidx = dsti[cur, pl.ds(f * LANE, LANE)]
                    dstl[cur, pl.ds(f * LANE, LANE)] = lidx
                    plsc.addupdate_scatter(cntv, [lidx], ones16)

                @plsc.parallel_loop(0, K, unroll=4)
                def _(e):
                    sums_edge(cur, e)

                @plsc.parallel_loop(0, K, step=LANE, unroll=2)
                def _(g):
                    stats_group(g)

                @plsc.parallel_loop(0, K, unroll=4)
                def _(e):
                    norm_edge(cur, e)

                pltpu.async_copy(bufa.at[cur], s_sh.at[dstl.at[cur]],
                                 ssc, add=True)

                @pl.when(jj + 2 < cpw)
                def _():
                    pltpu.async_copy(dsth.at[w, jj + 2], dsti.at[cur], sm)
                    pltpu.async_copy(srch.at[w, jj + 2], srci.at[cur], sm)

        wait_scat(1)
        pltpu.sync_copy(cntv, outc.at[w])
        plsc.subcore_barrier()
        pltpu.sync_copy(s_sh.at[pl.ds(sid * RPT, RPT)],
                        out.at[cid, pl.ds(sid * RPT, RPT)])

    return kern


CPW_FINE = 158
CPW_COARSE = 80
_SC_EDGE = {cpw: _make_sc_edge(cpw) for cpw in (CPW_FINE, CPW_COARSE)}


def _cnt_body(t_ref, o_ref):
    c = jnp.sum(t_ref[:], axis=0)
    o_ref[:] = c[:, None]


def _tc_cnt(t):
    return pl.pallas_call(
        _cnt_body,
        out_shape=jax.ShapeDtypeStruct((NPAD, 1), jnp.float32),
    )(t)


def _edge_blocks(edges, cpw):
    e = edges.shape[1]
    cap = NW * cpw * K
    src = edges[0].astype(jnp.int32)
    dst = edges[1].astype(jnp.int32)
    pad = jnp.full((cap - e,), N, jnp.int32)
    srcb = jnp.concatenate([src, pad]).reshape(NW, cpw, K)
    dstb = jnp.concatenate([dst, pad]).reshape(NW, cpw, K)
    return srcb, dstb


def _row(v):
    return v.reshape(1, -1)


def _gw_layer(p, h, srcb, dstb, cnt, zz, cpw):
    msg, upd = p["msg"], p["upd"]
    W1 = msg["l1"]["W"]
    A, B, Sself = _tc_pre(h, W1[:H], W1[H:], _row(msg["l1"]["b"]),
                          _row(msg["ln"]["g"]), _row(msg["ln"]["b"]))
    gb = jnp.stack([msg["ln"]["g"], msg["ln"]["b"]])
    S2, C2 = _SC_EDGE[cpw](A, B, dstb, srcb, zz, gb)
    if cnt is None:
        cnt = _tc_cnt(C2)
    U1 = upd["l1"]["W"]
    hnew = _tc_post(S2[0], S2[1], cnt, Sself, h,
                    msg["l2"]["W"], _row(msg["l2"]["b"]),
                    U1[:H], U1[H:], _row(upd["l1"]["b"]),
                    _row(upd["ln"]["g"]), _row(upd["ln"]["b"]),
                    upd["l2"]["W"], _row(upd["l2"]["b"]))
    return hnew, cnt


def kernel(x, fine_edges, coarse_edges, params):
    srcf, dstf = _edge_blocks(fine_edges, CPW_FINE)
    srcc, dstc = _edge_blocks(coarse_edges, CPW_COARSE)
    zz = jnp.zeros((NPAD, SW), jnp.float32)

    xp = jnp.pad(x, ((0, NPAD - N), (0, 0)))
    enc = params["enc"]
    h = _tc_enc(xp, enc["l1"]["W"], _row(enc["l1"]["b"]),
                _row(enc["ln"]["g"]), _row(enc["ln"]["b"]),
                enc["l2"]["W"], _row(enc["l2"]["b"]))

    hf, cf = h, None
    hc, cc = h, None
    order = [("f", 0), ("c", 0), ("f", 1), ("c", 1), ("f", 2), ("f", 3)]
    for branch, i in order:
        if branch == "f":
            hf, cf = _gw_layer(params["fine"][i], hf, srcf, dstf, cf, zz,
                               CPW_FINE)
        else:
            hc, cc = _gw_layer(params["coarse"][i], hc, srcc, dstc, cc, zz,
                               CPW_COARSE)

    Wm = params["mesh"]["W"]
    return _tc_dec(hf, hc, Wm[:H], Wm[H:], _row(params["mesh"]["b"]),
                   params["dec_l1"]["W"], _row(params["dec_l1"]["b"]),
                   params["dec_l2"]["W"], _row(params["dec_l2"]["b"]))
